# Initial kernel scaffold; baseline (speedup 1.0000x reference)
import functools
import jax
import jax.numpy as jnp
from jax import lax
from jax.experimental import pallas as pl
from jax.experimental.pallas import tpu as pltpu
from jax.experimental.pallas import tpu_sc as plsc

N = 50000
E = 800000
FIN = 19
D = 64

NC = 2
NS = 16
NW = NC * NS
L = 16

E_PAD = 819200
NZ = N + 16
NP4 = N + 16
NPS = 50176

EW_W = E_PAD // NW
B12 = 5120
NB12 = EW_W // B12
NG12 = B12 // L

EW_T = E_PAD // NS
BS = 2048
NBS = EW_T // BS
NGS = BS // L

RPT4 = NP4 // NS
RPTS = NPS // NS

_MESH = plsc.VectorSubcoreMesh(
    core_axis_name="c", subcore_axis_name="s", num_cores=NC, num_subcores=NS)


def _iota16():
  return lax.iota(jnp.int32, L)


def _sc_pass1_body(src_hbm, dst_hbm, ew_hbm, z_hbm, out_hbm,
                   z_v, src_v, dst_v, ew_v, res_v):
  c = lax.axis_index("c")
  s = lax.axis_index("s")
  wid = s * NC + c
  pltpu.sync_copy(z_hbm, z_v)
  base = wid * EW_W

  def batch_body(b, carry):
    s1, s2 = carry
    off = base + b * B12
    pltpu.sync_copy(src_hbm.at[pl.ds(off, B12)], src_v)
    pltpu.sync_copy(dst_hbm.at[pl.ds(off, B12)], dst_v)
    pltpu.sync_copy(ew_hbm.at[pl.ds(off, B12)], ew_v)

    def group_body(g, carry2):
      t1, t2 = carry2
      i0 = g * L
      sv = src_v[pl.ds(i0, L)]
      dv = dst_v[pl.ds(i0, L)]
      zj = plsc.load_gather(z_v, [sv])
      zi = plsc.load_gather(z_v, [dv])
      ewv = ew_v[pl.ds(i0, L)]
      dd = ewv - 0.5 * (zi + zj)
      ee = jnp.exp(-dd)
      wp = ee * ee - 2.0 * ee
      return (t1 + wp, t2 + wp * wp)

    return lax.fori_loop(0, NG12, group_body, (s1, s2))

  z16 = jnp.zeros((L,), jnp.float32)
  s1, s2 = lax.fori_loop(0, NB12, batch_body, (z16, z16))
  res_v[pl.ds(0, L)] = s1
  res_v[pl.ds(L, L)] = s2
  pltpu.sync_copy(res_v, out_hbm.at[wid])


_sc_pass1 = pl.kernel(
    _sc_pass1_body,
    out_type=jax.ShapeDtypeStruct((NW, 2 * L), jnp.float32),
    mesh=_MESH,
    scratch_types=[
        pltpu.VMEM((NZ,), jnp.float32),
        pltpu.VMEM((B12,), jnp.int32),
        pltpu.VMEM((B12,), jnp.int32),
        pltpu.VMEM((B12,), jnp.float32),
        pltpu.VMEM((2 * L,), jnp.float32),
    ],
)


def _sc_pass2_body(src_hbm, dst_hbm, ew_hbm, ea_hbm, z_hbm, ab_hbm, zz4_hbm,
                   wn_hbm, segp_hbm,
                   z_v, src_v, dst_v, ew_v, ea_v, wn_v, val4_v, ab_v, seg_acc):
  c = lax.axis_index("c")
  s = lax.axis_index("s")
  wid = s * NC + c
  pltpu.sync_copy(z_hbm, z_v)
  pltpu.sync_copy(ab_hbm, ab_v)
  pltpu.sync_copy(zz4_hbm, seg_acc.at[pl.ds(s * RPT4, RPT4)])
  plsc.subcore_barrier()

  av = ab_v[pl.ds(0, L)]
  bv = ab_v[pl.ds(L, L)]
  base = wid * EW_W
  ones = jnp.full((L,), 1.0, jnp.float32)

  def batch_body(b, carry):
    off = base + b * B12
    pltpu.sync_copy(src_hbm.at[pl.ds(off, B12)], src_v)
    pltpu.sync_copy(dst_hbm.at[pl.ds(off, B12)], dst_v)
    pltpu.sync_copy(ew_hbm.at[pl.ds(off, B12)], ew_v)
    pltpu.sync_copy(ea_hbm.at[pl.ds(off, B12)], ea_v)

    def group_body(g, carry2):
      i0 = g * L
      sv = src_v[pl.ds(i0, L)]
      dv = dst_v[pl.ds(i0, L)]
      zj = plsc.load_gather(z_v, [sv])
      zi = plsc.load_gather(z_v, [dv])
      ewv = ew_v[pl.ds(i0, L)]
      eav = ea_v[pl.ds(i0, L)]
      dd = ewv - 0.5 * (zi + zj)
      ee = jnp.exp(-dd)
      wp = ee * ee - 2.0 * ee
      wn = av * wp + bv
      wn_v[pl.ds(i0, L)] = wn
      ridx = _iota16() + i0
      plsc.store_scatter(val4_v, [ridx, jnp.zeros((L,), jnp.int32)], wn)
      plsc.store_scatter(val4_v, [ridx, jnp.full((L,), 1, jnp.int32)], ewv * wn)
      plsc.store_scatter(val4_v, [ridx, jnp.full((L,), 2, jnp.int32)], eav * wn)
      plsc.store_scatter(val4_v, [ridx, jnp.full((L,), 3, jnp.int32)], ones)
      return carry2

    lax.fori_loop(0, NG12, group_body, 0)
    pltpu.sync_copy(wn_v, wn_hbm.at[pl.ds(off, B12)])
    pltpu.sync_copy(val4_v, seg_acc.at[dst_v], add=True)
    return carry

  lax.fori_loop(0, NB12, batch_body, 0)
  plsc.subcore_barrier()
  pltpu.sync_copy(seg_acc.at[pl.ds(s * RPT4, RPT4)],
                  segp_hbm.at[c].at[pl.ds(s * RPT4, RPT4)])


_sc_pass2 = pl.kernel(
    _sc_pass2_body,
    out_type=[
        jax.ShapeDtypeStruct((E_PAD,), jnp.float32),
        jax.ShapeDtypeStruct((NC, NP4, 4), jnp.float32),
    ],
    mesh=_MESH,
    scratch_types=[
        pltpu.VMEM((NZ,), jnp.float32),
        pltpu.VMEM((B12,), jnp.int32),
        pltpu.VMEM((B12,), jnp.int32),
        pltpu.VMEM((B12,), jnp.float32),
        pltpu.VMEM((B12,), jnp.float32),
        pltpu.VMEM((B12,), jnp.float32),
        pltpu.VMEM((B12, 4), jnp.float32),
        pltpu.VMEM((2 * L,), jnp.float32),
        pltpu.VMEM_SHARED((NP4, 4), jnp.float32),
    ],
)


def _sc_spmm_body(src_hbm, dst_hbm, wn_hbm, tab_hbm, zz32_hbm,
                  g_hbm,
                  src_v, dst_v, wn_v, rows_v, acc, sem):
  c = lax.axis_index("c")
  s = lax.axis_index("s")
  pltpu.sync_copy(zz32_hbm, acc.at[pl.ds(s * RPTS, RPTS)])
  plsc.subcore_barrier()

  base = s * EW_T

  def batch_body(b, carry):
    off = base + b * BS
    pltpu.sync_copy(src_hbm.at[pl.ds(off, BS)], src_v)
    pltpu.sync_copy(dst_hbm.at[pl.ds(off, BS)], dst_v)
    pltpu.sync_copy(wn_hbm.at[pl.ds(off, BS)], wn_v)
    pltpu.async_copy(tab_hbm.at[c].at[src_v], rows_v, sem).wait()

    def group_body(g, carry2):
      i0 = g * L
      wv = wn_v[pl.ds(i0, L)]
      ridx = _iota16() + i0
      for col in range(D // 2):
        cidx = jnp.full((L,), col, jnp.int32)
        v = plsc.load_gather(rows_v, [ridx, cidx]) * wv
        plsc.store_scatter(rows_v, [ridx, cidx], v)
      return carry2

    lax.fori_loop(0, NGS, group_body, 0)
    pltpu.sync_copy(rows_v, acc.at[dst_v], add=True)
    return carry

  lax.fori_loop(0, NBS, batch_body, 0)
  plsc.subcore_barrier()
  pltpu.sync_copy(acc.at[pl.ds(s * RPTS, RPTS)],
                  g_hbm.at[c].at[pl.ds(s * RPTS, RPTS)])


_sc_spmm = pl.kernel(
    _sc_spmm_body,
    out_type=jax.ShapeDtypeStruct((NC, NPS, D // 2), jnp.float32),
    mesh=_MESH,
    scratch_types=[
        pltpu.VMEM((BS,), jnp.int32),
        pltpu.VMEM((BS,), jnp.int32),
        pltpu.VMEM((BS,), jnp.float32),
        pltpu.VMEM((BS, D // 2), jnp.float32),
        pltpu.VMEM_SHARED((NPS, D // 2), jnp.float32),
        pltpu.SemaphoreType.DMA,
    ],
)


_RB = 2000
_GRID = N // _RB


def _tc0_body(x_ref, w0t_ref, b0_ref, hs_ref):
  v = jnp.dot(x_ref[...], w0t_ref[...],
              preferred_element_type=jnp.float32) + b0_ref[...]
  h = jnp.where(v >= 0.0, v, 0.01 * v)
  hs_ref[0] = h[:, :D // 2]
  hs_ref[1] = h[:, D // 2:]


def _tc0(x, w0t, b0):
  return pl.pallas_call(
      _tc0_body,
      grid=(_GRID,),
      in_specs=[
          pl.BlockSpec((_RB, FIN), lambda i: (i, 0)),
          pl.BlockSpec((FIN, D), lambda i: (0, 0)),
          pl.BlockSpec((1, D), lambda i: (0, 0)),
      ],
      out_specs=pl.BlockSpec((NC, _RB, D // 2), lambda i: (0, i, 0)),
      out_shape=jax.ShapeDtypeStruct((NC, N, D // 2), jnp.float32),
  )(x, w0t, b0)


def _tc_dense_body(g_ref, hs_ref, segp_ref,
                   wat_ref, wbt_ref, wew_ref, wea_ref,
                   l2t_ref, b2_ref, l3t_ref, b3_ref,
                   wih_ref, bih_ref, whh_ref, bhh_ref,
                   o_ref):
  seg = segp_ref[0] + segp_ref[1]
  s_wn = seg[:, 0:1]
  s_ew = seg[:, 1:2]
  s_ea = seg[:, 2:3]
  cnt = seg[:, 3:4]
  g = jnp.concatenate([g_ref[0], g_ref[1]], axis=1)
  h = jnp.concatenate([hs_ref[0], hs_ref[1]], axis=1)

  dot = functools.partial(jnp.dot, preferred_element_type=jnp.float32)
  sums = (dot(h * s_wn, wat_ref[...]) + dot(g, wbt_ref[...])
          + s_ew * wew_ref[...] + s_ea * wea_ref[...])
  agg = sums / jnp.maximum(cnt, 1.0)
  m = dot(agg, l2t_ref[...]) + b2_ref[...]
  m = jnp.maximum(m, 0.0) + jnp.log1p(jnp.exp(-jnp.abs(m))) - 0.6931471805599453
  m = dot(m, l3t_ref[...]) + b3_ref[...]
  gi = dot(m, wih_ref[...]) + bih_ref[...]
  gh = dot(h, whh_ref[...]) + bhh_ref[...]
  r = jax.nn.sigmoid(gi[:, :D] + gh[:, :D])
  zt = jax.nn.sigmoid(gi[:, D:2 * D] + gh[:, D:2 * D])
  ng = jnp.tanh(gi[:, 2 * D:] + r * gh[:, 2 * D:])
  hn = (1.0 - zt) * ng + zt * h
  o_ref[0] = hn[:, :D // 2]
  o_ref[1] = hn[:, D // 2:]


def _tc_dense(g2, hs, segp, wat, wbt, wew, wea, l2t, b2, l3t, b3,
              wih, bih, whh, bhh):
  full = lambda shape: pl.BlockSpec(shape, lambda i, _s=shape: tuple(0 for _ in _s))
  return pl.pallas_call(
      _tc_dense_body,
      grid=(_GRID,),
      in_specs=[
          pl.BlockSpec((NC, _RB, D // 2), lambda i: (0, i, 0)),
          pl.BlockSpec((NC, _RB, D // 2), lambda i: (0, i, 0)),
          pl.BlockSpec((NC, _RB, 4), lambda i: (0, i, 0)),
          full((D, 2 * D)), full((D, 2 * D)), full((1, 2 * D)),
          full((1, 2 * D)),
          full((2 * D, 2 * D)), full((1, 2 * D)), full((2 * D, D)),
          full((1, D)),
          full((D, 3 * D)), full((1, 3 * D)), full((D, 3 * D)),
          full((1, 3 * D)),
      ],
      out_specs=pl.BlockSpec((NC, _RB, D // 2), lambda i: (0, i, 0)),
      out_shape=jax.ShapeDtypeStruct((NC, N, D // 2), jnp.float32),
  )(g2, hs, segp, wat, wbt, wew, wea, l2t, b2, l3t, b3, wih, bih, whh, bhh)


def kernel(x, edge_index, edge_weight, edge_attr, z, W0, b0, lin1_W,
           lin2_W, lin2_b, lin3_W, lin3_b, bn_gamma, bn_beta,
           gru_Wih, gru_Whh, gru_bih, gru_bhh):
  f32 = jnp.float32
  src = edge_index[0]
  dst = edge_index[1]
  npad = E_PAD - E
  srcp = jnp.concatenate([src, jnp.zeros((npad,), jnp.int32)])
  dstp = jnp.concatenate([dst, jnp.full((npad,), N, jnp.int32)])
  ewp = jnp.concatenate([edge_weight, jnp.full((npad,), 20.0, f32)])
  eap = jnp.concatenate([edge_attr[:, 0], jnp.zeros((npad,), f32)])
  zpad = jnp.concatenate([z[:, 0], jnp.zeros((NZ - N,), f32)])

  w0t = W0.T
  b0r = b0.reshape(1, D)
  wat = lin1_W[:, :D].T
  wbt = lin1_W[:, D:2 * D].T
  wew = lin1_W[:, 2 * D].reshape(1, 2 * D)
  wea = lin1_W[:, 2 * D + 1].reshape(1, 2 * D)
  l2t = lin2_W.T
  b2r = lin2_b.reshape(1, 2 * D)
  l3t = lin3_W.T
  b3r = lin3_b.reshape(1, D)
  wih = gru_Wih.T
  bih = gru_bih.reshape(1, 3 * D)
  whh = gru_Whh.T
  bhh = gru_bhh.reshape(1, 3 * D)

  zz4 = jnp.zeros((RPT4, 4), f32)
  zz32 = jnp.zeros((RPTS, D // 2), f32)

  part = _sc_pass1(srcp, dstp, ewp, zpad)
  s1 = jnp.sum(part[:, :L])
  s2 = jnp.sum(part[:, L:])
  mu = s1 / E
  var = s2 / E - mu * mu
  a = bn_gamma[0] / jnp.sqrt(var + 1e-5)
  b_ = bn_beta[0] - mu * a
  ab = jnp.concatenate([jnp.full((L,), a, f32), jnp.full((L,), b_, f32)])

  wn, segp = _sc_pass2(srcp, dstp, ewp, eap, zpad, ab, zz4)
  segp = segp[:, :N, :]

  hs = _tc0(x, w0t, b0r)

  for _ in range(3):
    g2 = _sc_spmm(srcp, dstp, wn, hs, zz32)
    g2 = g2[:, :N, :]
    hs = _tc_dense(g2, hs, segp, wat, wbt, wew, wea,
                   l2t, b2r, l3t, b3r, wih, bih, whh, bhh)

  return jnp.concatenate([hs[0], hs[1]], axis=1)


# speedup vs baseline: 5.0100x; 5.0100x over previous
import functools
import jax
import jax.numpy as jnp
from jax import lax
from jax.experimental import pallas as pl
from jax.experimental.pallas import tpu as pltpu
from jax.experimental.pallas import tpu_sc as plsc

N = 50000
E = 800000
FIN = 19
D = 64

NC = 2
NS = 16
NW = NC * NS
L = 16

E_PAD = 819200
NZ = N + 16
NPS = 50016

EW_W = E_PAD // NW
B12 = 5120
NB12 = EW_W // B12
NG12 = B12 // L

EW_T = E_PAD // NS
BS = 1024
NBS = EW_T // BS
NGS = BS // L

RPTS = NPS // NS
CHS = RPTS // 3

B2 = 1280
NB2 = EW_W // B2
NG2 = B2 // L

CW = 16
NCG = D // CW
NR = NCG // NC

_MESH = plsc.VectorSubcoreMesh(
    core_axis_name="c", subcore_axis_name="s", num_cores=NC, num_subcores=NS)

_SC_PARAMS = pltpu.CompilerParams(
    needs_layout_passes=False, use_tc_tiling_on_sc=False)


def _iota16():
  return lax.iota(jnp.int32, L)


def _read16(ref2d, g):
  row = jnp.full((L,), g // 8, jnp.int32)
  col = _iota16() + (g % 8) * L
  return plsc.load_gather(ref2d, [row, col])


def _sc_pass1_body(src2_hbm, dst2_hbm, ew_hbm, z_hbm, out_hbm,
                   z_v, src2_v, dst2_v, ew_v, res_v):
  c = lax.axis_index("c")
  s = lax.axis_index("s")
  wid = s * NC + c
  pltpu.sync_copy(z_hbm, z_v)
  base = wid * EW_W

  def batch_body(b, carry):
    s1, s2 = carry
    off = base + b * B12
    pltpu.sync_copy(src2_hbm.at[pl.ds(off // 128, B12 // 128)], src2_v)
    pltpu.sync_copy(dst2_hbm.at[pl.ds(off // 128, B12 // 128)], dst2_v)
    pltpu.sync_copy(ew_hbm.at[pl.ds(off, B12)], ew_v)

    def group_body(g, carry2):
      t1, t2 = carry2
      i0 = g * L
      sv = _read16(src2_v, g)
      dv = _read16(dst2_v, g)
      zj = plsc.load_gather(z_v, [sv])
      zi = plsc.load_gather(z_v, [dv])
      ewv = ew_v[pl.ds(i0, L)]
      dd = ewv - 0.5 * (zi + zj)
      ee = jnp.exp(-dd)
      wp = ee * ee - 2.0 * ee
      return (t1 + wp, t2 + wp * wp)

    return lax.fori_loop(0, NG12, group_body, (s1, s2))

  z16 = jnp.zeros((L,), jnp.float32)
  s1, s2 = lax.fori_loop(0, NB12, batch_body, (z16, z16))
  res_v[pl.ds(0, L)] = s1
  res_v[pl.ds(L, L)] = s2
  pltpu.sync_copy(res_v, out_hbm.at[pl.ds(wid * 128, 128)])


_sc_pass1 = pl.kernel(
    _sc_pass1_body,
    out_type=jax.ShapeDtypeStruct((NW * 128,), jnp.float32),
    mesh=_MESH,
    scratch_types=[
        pltpu.VMEM((NZ,), jnp.float32),
        pltpu.VMEM((B12 // 128, 128), jnp.int32),
        pltpu.VMEM((B12 // 128, 128), jnp.int32),
        pltpu.VMEM((B12,), jnp.float32),
        pltpu.VMEM((128,), jnp.float32),
    ],
    compiler_params=_SC_PARAMS,
)


def _sc_pass2_body(src2_hbm, dst2_hbm, ew_hbm, z_hbm, ab_hbm,
                   wn_hbm,
                   z_v, src2_v, dst2_v, ew_v, wn_v, ab_v):
  c = lax.axis_index("c")
  s = lax.axis_index("s")
  wid = s * NC + c
  pltpu.sync_copy(z_hbm, z_v)
  pltpu.sync_copy(ab_hbm, ab_v)

  av = ab_v[pl.ds(0, L)]
  bv = ab_v[pl.ds(L, L)]
  base = wid * EW_W

  def batch_body(b, carry):
    off = base + b * B2
    pltpu.sync_copy(src2_hbm.at[pl.ds(off // 128, B2 // 128)], src2_v)
    pltpu.sync_copy(dst2_hbm.at[pl.ds(off // 128, B2 // 128)], dst2_v)
    pltpu.sync_copy(ew_hbm.at[pl.ds(off, B2)], ew_v)

    def group_body(g, carry2):
      i0 = g * L
      sv = _read16(src2_v, g)
      dv = _read16(dst2_v, g)
      zj = plsc.load_gather(z_v, [sv])
      zi = plsc.load_gather(z_v, [dv])
      ewv = ew_v[pl.ds(i0, L)]
      dd = ewv - 0.5 * (zi + zj)
      ee = jnp.exp(-dd)
      wp = ee * ee - 2.0 * ee
      wn_v[pl.ds(i0, L)] = av * wp + bv
      return carry2

    lax.fori_loop(0, NG2, group_body, 0)
    pltpu.sync_copy(wn_v, wn_hbm.at[pl.ds(off, B2)])
    return carry

  lax.fori_loop(0, NB2, batch_body, 0)


_sc_pass2 = pl.kernel(
    _sc_pass2_body,
    out_type=jax.ShapeDtypeStruct((E_PAD,), jnp.float32),
    mesh=_MESH,
    scratch_types=[
        pltpu.VMEM((NZ,), jnp.float32),
        pltpu.VMEM((B2 // 128, 128), jnp.int32),
        pltpu.VMEM((B2 // 128, 128), jnp.int32),
        pltpu.VMEM((B2,), jnp.float32),
        pltpu.VMEM((B2,), jnp.float32),
        pltpu.VMEM((2 * L,), jnp.float32),
    ],
    compiler_params=_SC_PARAMS,
)


def _sc_spmm_body(src2_hbm, dst2_hbm, wn_hbm, ew_hbm, ea_hbm, tab_hbm,
                  zz32_hbm, zzv_hbm,
                  g_hbm, segp_hbm,
                  src2_v, dst2_v, wn_v, ew_v, ea_v, rows_v, val16_v, st_v,
                  acc, sem):
  c = lax.axis_index("c")
  s = lax.axis_index("s")
  base = s * EW_T
  pltpu.sync_copy(zz32_hbm, st_v)

  for r in range(NR):
    grp = c * NR + r
    for k in range(3):
      pltpu.sync_copy(st_v, acc.at[pl.ds(s * RPTS + k * CHS, CHS)])
    plsc.subcore_barrier()

    def batch_body(b, carry):
      off = base + b * BS
      offr = off // 128
      pltpu.sync_copy(src2_hbm.at[pl.ds(offr, BS // 128)], src2_v)
      pltpu.sync_copy(dst2_hbm.at[pl.ds(offr, BS // 128)], dst2_v)
      pltpu.sync_copy(wn_hbm.at[pl.ds(off, BS)], wn_v)
      gds = [
          pltpu.async_copy(tab_hbm.at[grp].at[src2_v.at[j]],
                           rows_v.at[pl.ds(j * 128, 128)], sem)
          for j in range(BS // 128)
      ]
      for d in gds:
        d.wait()

      def group_body(g, carry2):
        i0 = g * L
        wv = wn_v[pl.ds(i0, L)]
        ridx = _iota16() + i0
        for col in range(CW):
          cidx = jnp.full((L,), col, jnp.int32)
          v = plsc.load_gather(rows_v, [ridx, cidx]) * wv
          plsc.store_scatter(rows_v, [ridx, cidx], v)
        return carry2

      lax.fori_loop(0, NGS, group_body, 0)
      sds = [
          pltpu.async_copy(rows_v.at[pl.ds(j * 128, 128)],
                           acc.at[dst2_v.at[j]], sem, add=True)
          for j in range(BS // 128)
      ]
      for d in sds:
        d.wait()
      return carry

    lax.fori_loop(0, NBS, batch_body, 0)
    plsc.subcore_barrier()
    for k in range(3):
      r0 = s * RPTS + k * CHS
      pltpu.sync_copy(acc.at[pl.ds(r0, CHS)], st_v)
      pltpu.sync_copy(st_v, g_hbm.at[grp].at[pl.ds(r0, CHS)])
    plsc.subcore_barrier()
    pltpu.sync_copy(zz32_hbm, st_v)

  pltpu.sync_copy(zzv_hbm, val16_v)
  for k in range(3):
    pltpu.sync_copy(st_v, acc.at[pl.ds(s * RPTS + k * CHS, CHS)])
  plsc.subcore_barrier()
  ones = jnp.full((L,), 1.0, jnp.float32)
  sbase = c * (E_PAD // 2) + s * (E_PAD // 2 // NS)

  def seg_batch(b, carry):
    off = sbase + b * BS
    pltpu.sync_copy(dst2_hbm.at[pl.ds(off // 128, BS // 128)], dst2_v)
    pltpu.sync_copy(wn_hbm.at[pl.ds(off, BS)], wn_v)
    pltpu.sync_copy(ew_hbm.at[pl.ds(off, BS)], ew_v)
    pltpu.sync_copy(ea_hbm.at[pl.ds(off, BS)], ea_v)

    def seg_group(g, carry2):
      i0 = g * L
      wn = wn_v[pl.ds(i0, L)]
      ewv = ew_v[pl.ds(i0, L)]
      eav = ea_v[pl.ds(i0, L)]
      ridx = _iota16() + i0
      plsc.store_scatter(val16_v, [ridx, jnp.zeros((L,), jnp.int32)], wn)
      plsc.store_scatter(val16_v, [ridx, jnp.full((L,), 1, jnp.int32)],
                         ewv * wn)
      plsc.store_scatter(val16_v, [ridx, jnp.full((L,), 2, jnp.int32)],
                         eav * wn)
      plsc.store_scatter(val16_v, [ridx, jnp.full((L,), 3, jnp.int32)], ones)
      return carry2

    lax.fori_loop(0, NGS, seg_group, 0)
    descs = [
        pltpu.async_copy(val16_v.at[pl.ds(j * 128, 128)],
                         acc.at[dst2_v.at[j]], sem, add=True)
        for j in range(BS // 128)
    ]
    for d in descs:
      d.wait()
    return carry

  lax.fori_loop(0, E_PAD // 2 // NS // BS, seg_batch, 0)
  plsc.subcore_barrier()
  for k in range(3):
    r0 = s * RPTS + k * CHS
    pltpu.sync_copy(acc.at[pl.ds(r0, CHS)], st_v)
    pltpu.sync_copy(st_v, segp_hbm.at[c].at[pl.ds(r0, CHS)])


_sc_spmm = pl.kernel(
    _sc_spmm_body,
    out_type=[
        jax.ShapeDtypeStruct((NCG, NPS, CW), jnp.float32),
        jax.ShapeDtypeStruct((NC, NPS, CW), jnp.float32),
    ],
    mesh=_MESH,
    scratch_types=[
        pltpu.VMEM((BS // 128, 128), jnp.int32),
        pltpu.VMEM((BS // 128, 128), jnp.int32),
        pltpu.VMEM((BS,), jnp.float32),
        pltpu.VMEM((BS,), jnp.float32),
        pltpu.VMEM((BS,), jnp.float32),
        pltpu.VMEM((BS, CW), jnp.float32),
        pltpu.VMEM((BS, CW), jnp.float32),
        pltpu.VMEM((CHS, CW), jnp.float32),
        pltpu.VMEM_SHARED((NPS, CW), jnp.float32),
        pltpu.SemaphoreType.DMA,
    ],
    compiler_params=_SC_PARAMS,
)


_RB = 2000
_GRID = N // _RB


def _tc0_body(x_ref, w0t_ref, b0_ref, hs_ref):
  v = jnp.dot(x_ref[...], w0t_ref[...],
              preferred_element_type=jnp.float32) + b0_ref[...]
  h = jnp.where(v >= 0.0, v, 0.01 * v)
  for k in range(NCG):
    hs_ref[k] = h[:, k * CW:(k + 1) * CW]


def _tc0(x, w0t, b0):
  return pl.pallas_call(
      _tc0_body,
      grid=(_GRID,),
      in_specs=[
          pl.BlockSpec((_RB, FIN), lambda i: (i, 0)),
          pl.BlockSpec((FIN, D), lambda i: (0, 0)),
          pl.BlockSpec((1, D), lambda i: (0, 0)),
      ],
      out_specs=pl.BlockSpec((NCG, _RB, CW), lambda i: (0, i, 0)),
      out_shape=jax.ShapeDtypeStruct((NCG, N, CW), jnp.float32),
  )(x, w0t, b0)


def _tc_dense_body(g_ref, hs_ref, seg_ref,
                   wat_ref, wbt_ref, wew_ref, wea_ref,
                   l2t_ref, b2_ref, l3t_ref, b3_ref,
                   wih_ref, bih_ref, whh_ref, bhh_ref,
                   o_ref):
  seg = seg_ref[0][:, :4] + seg_ref[1][:, :4]
  s_wn = seg[:, 0:1]
  s_ew = seg[:, 1:2]
  s_ea = seg[:, 2:3]
  cnt = seg[:, 3:4]
  g = jnp.concatenate([g_ref[k] for k in range(NCG)], axis=1)
  h = jnp.concatenate([hs_ref[k] for k in range(NCG)], axis=1)

  dot = functools.partial(jnp.dot, preferred_element_type=jnp.float32)
  sums = (dot(h * s_wn, wat_ref[...]) + dot(g, wbt_ref[...])
          + s_ew * wew_ref[...] + s_ea * wea_ref[...])
  agg = sums / jnp.maximum(cnt, 1.0)
  m = dot(agg, l2t_ref[...]) + b2_ref[...]
  m = jnp.maximum(m, 0.0) + jnp.log1p(jnp.exp(-jnp.abs(m))) - 0.6931471805599453
  m = dot(m, l3t_ref[...]) + b3_ref[...]
  gi = dot(m, wih_ref[...]) + bih_ref[...]
  gh = dot(h, whh_ref[...]) + bhh_ref[...]
  r = jax.nn.sigmoid(gi[:, :D] + gh[:, :D])
  zt = jax.nn.sigmoid(gi[:, D:2 * D] + gh[:, D:2 * D])
  ng = jnp.tanh(gi[:, 2 * D:] + r * gh[:, 2 * D:])
  hn = (1.0 - zt) * ng + zt * h
  for k in range(NCG):
    o_ref[k] = hn[:, k * CW:(k + 1) * CW]


def _tc_dense(g2, hs, segp, wat, wbt, wew, wea, l2t, b2, l3t, b3,
              wih, bih, whh, bhh):
  full = lambda shape: pl.BlockSpec(shape, lambda i, _s=shape: tuple(0 for _ in _s))
  return pl.pallas_call(
      _tc_dense_body,
      grid=(_GRID,),
      in_specs=[
          pl.BlockSpec((NCG, _RB, CW), lambda i: (0, i, 0)),
          pl.BlockSpec((NCG, _RB, CW), lambda i: (0, i, 0)),
          pl.BlockSpec((NC, _RB, CW), lambda i: (0, i, 0)),
          full((D, 2 * D)), full((D, 2 * D)), full((1, 2 * D)),
          full((1, 2 * D)),
          full((2 * D, 2 * D)), full((1, 2 * D)), full((2 * D, D)),
          full((1, D)),
          full((D, 3 * D)), full((1, 3 * D)), full((D, 3 * D)),
          full((1, 3 * D)),
      ],
      out_specs=pl.BlockSpec((NCG, _RB, CW), lambda i: (0, i, 0)),
      out_shape=jax.ShapeDtypeStruct((NCG, N, CW), jnp.float32),
  )(g2, hs, segp, wat, wbt, wew, wea, l2t, b2, l3t, b3, wih, bih, whh, bhh)


def kernel(x, edge_index, edge_weight, edge_attr, z, W0, b0, lin1_W,
           lin2_W, lin2_b, lin3_W, lin3_b, bn_gamma, bn_beta,
           gru_Wih, gru_Whh, gru_bih, gru_bhh):
  f32 = jnp.float32
  src = edge_index[0]
  dst = edge_index[1]
  npad = E_PAD - E
  srcp = jnp.concatenate([src, jnp.zeros((npad,), jnp.int32)])
  dstp = jnp.concatenate([dst, jnp.full((npad,), N, jnp.int32)])
  srcp2 = srcp.reshape(E_PAD // 128, 128)
  dstp2 = dstp.reshape(E_PAD // 128, 128)
  ewp = jnp.concatenate([edge_weight, jnp.full((npad,), 20.0, f32)])
  eap = jnp.concatenate([edge_attr[:, 0], jnp.zeros((npad,), f32)])
  zpad = jnp.concatenate([z[:, 0], jnp.zeros((NZ - N,), f32)])

  w0t = W0.T
  b0r = b0.reshape(1, D)
  wat = lin1_W[:, :D].T
  wbt = lin1_W[:, D:2 * D].T
  wew = lin1_W[:, 2 * D].reshape(1, 2 * D)
  wea = lin1_W[:, 2 * D + 1].reshape(1, 2 * D)
  l2t = lin2_W.T
  b2r = lin2_b.reshape(1, 2 * D)
  l3t = lin3_W.T
  b3r = lin3_b.reshape(1, D)
  wih = gru_Wih.T
  bih = gru_bih.reshape(1, 3 * D)
  whh = gru_Whh.T
  bhh = gru_bhh.reshape(1, 3 * D)

  zzv = jnp.zeros((BS, CW), f32)
  zz32 = jnp.zeros((CHS, CW), f32)

  part = _sc_pass1(srcp2, dstp2, ewp, zpad).reshape(NW, 128)
  s1 = jnp.sum(part[:, :L])
  s2 = jnp.sum(part[:, L:2 * L])
  mu = s1 / E
  var = s2 / E - mu * mu
  a = bn_gamma[0] / jnp.sqrt(var + 1e-5)
  b_ = bn_beta[0] - mu * a
  ab = jnp.concatenate([jnp.full((L,), a, f32), jnp.full((L,), b_, f32)])

  wn = _sc_pass2(srcp2, dstp2, ewp, zpad, ab)

  hs = _tc0(x, w0t, b0r)

  for _ in range(3):
    g2, segp = _sc_spmm(srcp2, dstp2, wn, ewp, eap, hs, zz32, zzv)
    hs = _tc_dense(g2, hs, segp, wat, wbt, wew, wea,
                   l2t, b2r, l3t, b3r, wih, bih, whh, bhh)

  return jnp.concatenate([hs[k] for k in range(NCG)], axis=1)


# speedup vs baseline: 6.3061x; 1.2587x over previous
import functools
import jax
import jax.numpy as jnp
from jax import lax
from jax.experimental import pallas as pl
from jax.experimental.pallas import tpu as pltpu
from jax.experimental.pallas import tpu_sc as plsc

N = 50000
E = 800000
FIN = 19
D = 64

NC = 2
NS = 16
NW = NC * NS
L = 16

E_PAD = 819200
NZ = N + 16
NPS = 50016

EW_W = E_PAD // NW
B12 = 5120
NB12 = EW_W // B12
NG12 = B12 // L

EW_T = E_PAD // NS
BS = 1024
NBS = EW_T // BS
NGS = BS // L

RPTS = NPS // NS
CHS = RPTS // 3

B2 = 1280
NB2 = EW_W // B2
NG2 = B2 // L

CW = 16
NCG = D // CW
NR = NCG // NC

_MESH = plsc.VectorSubcoreMesh(
    core_axis_name="c", subcore_axis_name="s", num_cores=NC, num_subcores=NS)

_SC_PARAMS = pltpu.CompilerParams(
    needs_layout_passes=False, use_tc_tiling_on_sc=False)


def _iota16():
  return lax.iota(jnp.int32, L)


def _read16(ref2d, g):
  row = jnp.full((L,), g // 8, jnp.int32)
  col = _iota16() + (g % 8) * L
  return plsc.load_gather(ref2d, [row, col])


def _sc_pass1_body(src2_hbm, dst2_hbm, ew_hbm, z_hbm, out_hbm,
                   z_v, src2_v, dst2_v, ew_v, res_v):
  c = lax.axis_index("c")
  s = lax.axis_index("s")
  wid = s * NC + c
  pltpu.sync_copy(z_hbm, z_v)
  base = wid * EW_W

  def batch_body(b, carry):
    s1, s2 = carry
    off = base + b * B12
    pltpu.sync_copy(src2_hbm.at[pl.ds(off // 128, B12 // 128)], src2_v)
    pltpu.sync_copy(dst2_hbm.at[pl.ds(off // 128, B12 // 128)], dst2_v)
    pltpu.sync_copy(ew_hbm.at[pl.ds(off, B12)], ew_v)

    def group_body(g, carry2):
      t1, t2 = carry2
      i0 = g * L
      sv = _read16(src2_v, g)
      dv = _read16(dst2_v, g)
      zj = plsc.load_gather(z_v, [sv])
      zi = plsc.load_gather(z_v, [dv])
      ewv = ew_v[pl.ds(i0, L)]
      dd = ewv - 0.5 * (zi + zj)
      ee = jnp.exp(-dd)
      wp = ee * ee - 2.0 * ee
      return (t1 + wp, t2 + wp * wp)

    return lax.fori_loop(0, NG12, group_body, (s1, s2))

  z16 = jnp.zeros((L,), jnp.float32)
  s1, s2 = lax.fori_loop(0, NB12, batch_body, (z16, z16))
  res_v[pl.ds(0, L)] = s1
  res_v[pl.ds(L, L)] = s2
  pltpu.sync_copy(res_v, out_hbm.at[pl.ds(wid * 128, 128)])


_sc_pass1 = pl.kernel(
    _sc_pass1_body,
    out_type=jax.ShapeDtypeStruct((NW * 128,), jnp.float32),
    mesh=_MESH,
    scratch_types=[
        pltpu.VMEM((NZ,), jnp.float32),
        pltpu.VMEM((B12 // 128, 128), jnp.int32),
        pltpu.VMEM((B12 // 128, 128), jnp.int32),
        pltpu.VMEM((B12,), jnp.float32),
        pltpu.VMEM((128,), jnp.float32),
    ],
    compiler_params=_SC_PARAMS,
)


def _sc_pass2_body(src2_hbm, dst2_hbm, ew_hbm, z_hbm, ab_hbm,
                   wn_hbm,
                   z_v, src2_v, dst2_v, ew_v, wn_v, ab_v):
  c = lax.axis_index("c")
  s = lax.axis_index("s")
  wid = s * NC + c
  pltpu.sync_copy(z_hbm, z_v)
  pltpu.sync_copy(ab_hbm, ab_v)

  av = ab_v[pl.ds(0, L)]
  bv = ab_v[pl.ds(L, L)]
  base = wid * EW_W

  def batch_body(b, carry):
    off = base + b * B2
    pltpu.sync_copy(src2_hbm.at[pl.ds(off // 128, B2 // 128)], src2_v)
    pltpu.sync_copy(dst2_hbm.at[pl.ds(off // 128, B2 // 128)], dst2_v)
    pltpu.sync_copy(ew_hbm.at[pl.ds(off, B2)], ew_v)

    @plsc.parallel_loop(0, NG2, unroll=2)
    def _wn_group(g):
      i0 = g * L
      sv = _read16(src2_v, g)
      dv = _read16(dst2_v, g)
      zj = plsc.load_gather(z_v, [sv])
      zi = plsc.load_gather(z_v, [dv])
      ewv = ew_v[pl.ds(i0, L)]
      dd = ewv - 0.5 * (zi + zj)
      ee = jnp.exp(-dd)
      wp = ee * ee - 2.0 * ee
      wn_v[pl.ds(i0, L)] = av * wp + bv
    pltpu.sync_copy(wn_v, wn_hbm.at[pl.ds(off, B2)])
    return carry

  lax.fori_loop(0, NB2, batch_body, 0)


_sc_pass2 = pl.kernel(
    _sc_pass2_body,
    out_type=jax.ShapeDtypeStruct((E_PAD,), jnp.float32),
    mesh=_MESH,
    scratch_types=[
        pltpu.VMEM((NZ,), jnp.float32),
        pltpu.VMEM((B2 // 128, 128), jnp.int32),
        pltpu.VMEM((B2 // 128, 128), jnp.int32),
        pltpu.VMEM((B2,), jnp.float32),
        pltpu.VMEM((B2,), jnp.float32),
        pltpu.VMEM((2 * L,), jnp.float32),
    ],
    compiler_params=_SC_PARAMS,
)


def _sc_spmm_body(src2_hbm, dst2_hbm, wn_hbm, ew_hbm, ea_hbm, tab_hbm,
                  zz32_hbm, zzv_hbm,
                  g_hbm, segp_hbm,
                  src2_v, dst2_v, wn_v, ew_v, ea_v, rows_v, val16_v, st_v,
                  acc, sem):
  c = lax.axis_index("c")
  s = lax.axis_index("s")
  base = s * EW_T
  pltpu.sync_copy(zz32_hbm, st_v)

  for r in range(NR):
    grp = c * NR + r
    for k in range(3):
      pltpu.sync_copy(st_v, acc.at[pl.ds(s * RPTS + k * CHS, CHS)])
    plsc.subcore_barrier()

    def batch_body(b, carry):
      off = base + b * BS
      offr = off // 128
      pltpu.sync_copy(src2_hbm.at[pl.ds(offr, BS // 128)], src2_v)
      pltpu.sync_copy(dst2_hbm.at[pl.ds(offr, BS // 128)], dst2_v)
      pltpu.sync_copy(wn_hbm.at[pl.ds(off, BS)], wn_v)
      gds = [
          pltpu.async_copy(tab_hbm.at[grp].at[src2_v.at[j]],
                           rows_v.at[pl.ds(j * 128, 128)], sem)
          for j in range(BS // 128)
      ]
      for d in gds:
        d.wait()

      @plsc.parallel_loop(0, NGS, unroll=4)
      def _scale(g):
        i0 = g * L
        wv = wn_v[pl.ds(i0, L)]
        ridx = _iota16() + i0
        for col in range(CW):
          cidx = jnp.full((L,), col, jnp.int32)
          v = plsc.load_gather(rows_v, [ridx, cidx]) * wv
          plsc.store_scatter(rows_v, [ridx, cidx], v)
      sds = [
          pltpu.async_copy(rows_v.at[pl.ds(j * 128, 128)],
                           acc.at[dst2_v.at[j]], sem, add=True)
          for j in range(BS // 128)
      ]
      for d in sds:
        d.wait()
      return carry

    lax.fori_loop(0, NBS, batch_body, 0)
    plsc.subcore_barrier()
    for k in range(3):
      r0 = s * RPTS + k * CHS
      pltpu.sync_copy(acc.at[pl.ds(r0, CHS)], st_v)
      pltpu.sync_copy(st_v, g_hbm.at[grp].at[pl.ds(r0, CHS)])
    plsc.subcore_barrier()
    pltpu.sync_copy(zz32_hbm, st_v)

  pltpu.sync_copy(zzv_hbm, val16_v)
  for k in range(3):
    pltpu.sync_copy(st_v, acc.at[pl.ds(s * RPTS + k * CHS, CHS)])
  plsc.subcore_barrier()
  ones = jnp.full((L,), 1.0, jnp.float32)
  sbase = c * (E_PAD // 2) + s * (E_PAD // 2 // NS)

  def seg_batch(b, carry):
    off = sbase + b * BS
    pltpu.sync_copy(dst2_hbm.at[pl.ds(off // 128, BS // 128)], dst2_v)
    pltpu.sync_copy(wn_hbm.at[pl.ds(off, BS)], wn_v)
    pltpu.sync_copy(ew_hbm.at[pl.ds(off, BS)], ew_v)
    pltpu.sync_copy(ea_hbm.at[pl.ds(off, BS)], ea_v)

    @plsc.parallel_loop(0, NGS, unroll=4)
    def _seg_group(g):
      i0 = g * L
      wn = wn_v[pl.ds(i0, L)]
      ewv = ew_v[pl.ds(i0, L)]
      eav = ea_v[pl.ds(i0, L)]
      ridx = _iota16() + i0
      plsc.store_scatter(val16_v, [ridx, jnp.zeros((L,), jnp.int32)], wn)
      plsc.store_scatter(val16_v, [ridx, jnp.full((L,), 1, jnp.int32)],
                         ewv * wn)
      plsc.store_scatter(val16_v, [ridx, jnp.full((L,), 2, jnp.int32)],
                         eav * wn)
      plsc.store_scatter(val16_v, [ridx, jnp.full((L,), 3, jnp.int32)], ones)
    descs = [
        pltpu.async_copy(val16_v.at[pl.ds(j * 128, 128)],
                         acc.at[dst2_v.at[j]], sem, add=True)
        for j in range(BS // 128)
    ]
    for d in descs:
      d.wait()
    return carry

  lax.fori_loop(0, E_PAD // 2 // NS // BS, seg_batch, 0)
  plsc.subcore_barrier()
  for k in range(3):
    r0 = s * RPTS + k * CHS
    pltpu.sync_copy(acc.at[pl.ds(r0, CHS)], st_v)
    pltpu.sync_copy(st_v, segp_hbm.at[c].at[pl.ds(r0, CHS)])


_sc_spmm = pl.kernel(
    _sc_spmm_body,
    out_type=[
        jax.ShapeDtypeStruct((NCG, NPS, CW), jnp.float32),
        jax.ShapeDtypeStruct((NC, NPS, CW), jnp.float32),
    ],
    mesh=_MESH,
    scratch_types=[
        pltpu.VMEM((BS // 128, 128), jnp.int32),
        pltpu.VMEM((BS // 128, 128), jnp.int32),
        pltpu.VMEM((BS,), jnp.float32),
        pltpu.VMEM((BS,), jnp.float32),
        pltpu.VMEM((BS,), jnp.float32),
        pltpu.VMEM((BS, CW), jnp.float32),
        pltpu.VMEM((BS, CW), jnp.float32),
        pltpu.VMEM((CHS, CW), jnp.float32),
        pltpu.VMEM_SHARED((NPS, CW), jnp.float32),
        pltpu.SemaphoreType.DMA,
    ],
    compiler_params=_SC_PARAMS,
)


_RB = 2000
_GRID = N // _RB


def _tc0_body(x_ref, w0t_ref, b0_ref, hs_ref):
  v = jnp.dot(x_ref[...], w0t_ref[...],
              preferred_element_type=jnp.float32) + b0_ref[...]
  h = jnp.where(v >= 0.0, v, 0.01 * v)
  for k in range(NCG):
    hs_ref[k] = h[:, k * CW:(k + 1) * CW]


def _tc0(x, w0t, b0):
  return pl.pallas_call(
      _tc0_body,
      grid=(_GRID,),
      in_specs=[
          pl.BlockSpec((_RB, FIN), lambda i: (i, 0)),
          pl.BlockSpec((FIN, D), lambda i: (0, 0)),
          pl.BlockSpec((1, D), lambda i: (0, 0)),
      ],
      out_specs=pl.BlockSpec((NCG, _RB, CW), lambda i: (0, i, 0)),
      out_shape=jax.ShapeDtypeStruct((NCG, N, CW), jnp.float32),
  )(x, w0t, b0)


def _tc_dense_body(g_ref, hs_ref, seg_ref,
                   wat_ref, wbt_ref, wew_ref, wea_ref,
                   l2t_ref, b2_ref, l3t_ref, b3_ref,
                   wih_ref, bih_ref, whh_ref, bhh_ref,
                   o_ref):
  seg = seg_ref[0][:, :4] + seg_ref[1][:, :4]
  s_wn = seg[:, 0:1]
  s_ew = seg[:, 1:2]
  s_ea = seg[:, 2:3]
  cnt = seg[:, 3:4]
  g = jnp.concatenate([g_ref[k] for k in range(NCG)], axis=1)
  h = jnp.concatenate([hs_ref[k] for k in range(NCG)], axis=1)

  dot = functools.partial(jnp.dot, preferred_element_type=jnp.float32)
  sums = (dot(h * s_wn, wat_ref[...]) + dot(g, wbt_ref[...])
          + s_ew * wew_ref[...] + s_ea * wea_ref[...])
  agg = sums / jnp.maximum(cnt, 1.0)
  m = dot(agg, l2t_ref[...]) + b2_ref[...]
  m = jnp.maximum(m, 0.0) + jnp.log1p(jnp.exp(-jnp.abs(m))) - 0.6931471805599453
  m = dot(m, l3t_ref[...]) + b3_ref[...]
  gi = dot(m, wih_ref[...]) + bih_ref[...]
  gh = dot(h, whh_ref[...]) + bhh_ref[...]
  r = jax.nn.sigmoid(gi[:, :D] + gh[:, :D])
  zt = jax.nn.sigmoid(gi[:, D:2 * D] + gh[:, D:2 * D])
  ng = jnp.tanh(gi[:, 2 * D:] + r * gh[:, 2 * D:])
  hn = (1.0 - zt) * ng + zt * h
  for k in range(NCG):
    o_ref[k] = hn[:, k * CW:(k + 1) * CW]


def _tc_dense(g2, hs, segp, wat, wbt, wew, wea, l2t, b2, l3t, b3,
              wih, bih, whh, bhh):
  full = lambda shape: pl.BlockSpec(shape, lambda i, _s=shape: tuple(0 for _ in _s))
  return pl.pallas_call(
      _tc_dense_body,
      grid=(_GRID,),
      in_specs=[
          pl.BlockSpec((NCG, _RB, CW), lambda i: (0, i, 0)),
          pl.BlockSpec((NCG, _RB, CW), lambda i: (0, i, 0)),
          pl.BlockSpec((NC, _RB, CW), lambda i: (0, i, 0)),
          full((D, 2 * D)), full((D, 2 * D)), full((1, 2 * D)),
          full((1, 2 * D)),
          full((2 * D, 2 * D)), full((1, 2 * D)), full((2 * D, D)),
          full((1, D)),
          full((D, 3 * D)), full((1, 3 * D)), full((D, 3 * D)),
          full((1, 3 * D)),
      ],
      out_specs=pl.BlockSpec((NCG, _RB, CW), lambda i: (0, i, 0)),
      out_shape=jax.ShapeDtypeStruct((NCG, N, CW), jnp.float32),
  )(g2, hs, segp, wat, wbt, wew, wea, l2t, b2, l3t, b3, wih, bih, whh, bhh)


def kernel(x, edge_index, edge_weight, edge_attr, z, W0, b0, lin1_W,
           lin2_W, lin2_b, lin3_W, lin3_b, bn_gamma, bn_beta,
           gru_Wih, gru_Whh, gru_bih, gru_bhh):
  f32 = jnp.float32
  src = edge_index[0]
  dst = edge_index[1]
  npad = E_PAD - E
  srcp = jnp.concatenate([src, jnp.zeros((npad,), jnp.int32)])
  dstp = jnp.concatenate([dst, jnp.full((npad,), N, jnp.int32)])
  srcp2 = srcp.reshape(E_PAD // 128, 128)
  dstp2 = dstp.reshape(E_PAD // 128, 128)
  ewp = jnp.concatenate([edge_weight, jnp.full((npad,), 20.0, f32)])
  eap = jnp.concatenate([edge_attr[:, 0], jnp.zeros((npad,), f32)])
  zpad = jnp.concatenate([z[:, 0], jnp.zeros((NZ - N,), f32)])

  w0t = W0.T
  b0r = b0.reshape(1, D)
  wat = lin1_W[:, :D].T
  wbt = lin1_W[:, D:2 * D].T
  wew = lin1_W[:, 2 * D].reshape(1, 2 * D)
  wea = lin1_W[:, 2 * D + 1].reshape(1, 2 * D)
  l2t = lin2_W.T
  b2r = lin2_b.reshape(1, 2 * D)
  l3t = lin3_W.T
  b3r = lin3_b.reshape(1, D)
  wih = gru_Wih.T
  bih = gru_bih.reshape(1, 3 * D)
  whh = gru_Whh.T
  bhh = gru_bhh.reshape(1, 3 * D)

  zzv = jnp.zeros((BS, CW), f32)
  zz32 = jnp.zeros((CHS, CW), f32)

  part = _sc_pass1(srcp2, dstp2, ewp, zpad).reshape(NW, 128)
  s1 = jnp.sum(part[:, :L])
  s2 = jnp.sum(part[:, L:2 * L])
  mu = s1 / E
  var = s2 / E - mu * mu
  a = bn_gamma[0] / jnp.sqrt(var + 1e-5)
  b_ = bn_beta[0] - mu * a
  ab = jnp.concatenate([jnp.full((L,), a, f32), jnp.full((L,), b_, f32)])

  wn = _sc_pass2(srcp2, dstp2, ewp, zpad, ab)

  hs = _tc0(x, w0t, b0r)

  for _ in range(3):
    g2, segp = _sc_spmm(srcp2, dstp2, wn, ewp, eap, hs, zz32, zzv)
    hs = _tc_dense(g2, hs, segp, wat, wbt, wew, wea,
                   l2t, b2r, l3t, b3r, wih, bih, whh, bhh)

  return jnp.concatenate([hs[k] for k in range(NCG)], axis=1)


# speedup vs baseline: 6.6636x; 1.0567x over previous
import functools
import jax
import jax.numpy as jnp
from jax import lax
from jax.experimental import pallas as pl
from jax.experimental.pallas import tpu as pltpu
from jax.experimental.pallas import tpu_sc as plsc

N = 50000
E = 800000
FIN = 19
D = 64

NC = 2
NS = 16
NW = NC * NS
L = 16

E_PAD = 819200
NZ = N + 16
NPS = 50016

EW_W = E_PAD // NW
B12 = 5120
NB12 = EW_W // B12
NG12 = B12 // L

EW_T = E_PAD // NS
BS = 1024
NBS = EW_T // BS
NGS = BS // L

RPTS = NPS // NS
CHS = RPTS // 3

B2 = 1280
NB2 = EW_W // B2
NG2 = B2 // L

CW = 16
NCG = D // CW
NR = NCG // NC

_MESH = plsc.VectorSubcoreMesh(
    core_axis_name="c", subcore_axis_name="s", num_cores=NC, num_subcores=NS)

_SC_PARAMS = pltpu.CompilerParams(
    needs_layout_passes=False, use_tc_tiling_on_sc=False)


def _iota16():
  return lax.iota(jnp.int32, L)


def _read16(ref2d, g):
  row = jnp.full((L,), g // 8, jnp.int32)
  col = _iota16() + (g % 8) * L
  return plsc.load_gather(ref2d, [row, col])


def _sc_pass1_body(src2_hbm, dst2_hbm, ew_hbm, z_hbm, out_hbm,
                   z_v, src2_v, dst2_v, ew_v, res_v):
  c = lax.axis_index("c")
  s = lax.axis_index("s")
  wid = s * NC + c
  pltpu.sync_copy(z_hbm, z_v)
  base = wid * EW_W

  def batch_body(b, carry):
    s1, s2 = carry
    off = base + b * B12
    pltpu.sync_copy(src2_hbm.at[pl.ds(off // 128, B12 // 128)], src2_v)
    pltpu.sync_copy(dst2_hbm.at[pl.ds(off // 128, B12 // 128)], dst2_v)
    pltpu.sync_copy(ew_hbm.at[pl.ds(off, B12)], ew_v)

    def group_body(g, carry2):
      t1, t2 = carry2
      i0 = g * L
      sv = _read16(src2_v, g)
      dv = _read16(dst2_v, g)
      zj = plsc.load_gather(z_v, [sv])
      zi = plsc.load_gather(z_v, [dv])
      ewv = ew_v[pl.ds(i0, L)]
      dd = ewv - 0.5 * (zi + zj)
      ee = jnp.exp(-dd)
      wp = ee * ee - 2.0 * ee
      return (t1 + wp, t2 + wp * wp)

    return lax.fori_loop(0, NG12, group_body, (s1, s2))

  z16 = jnp.zeros((L,), jnp.float32)
  s1, s2 = lax.fori_loop(0, NB12, batch_body, (z16, z16))
  res_v[pl.ds(0, L)] = s1
  res_v[pl.ds(L, L)] = s2
  pltpu.sync_copy(res_v, out_hbm.at[pl.ds(wid * 128, 128)])


_sc_pass1 = pl.kernel(
    _sc_pass1_body,
    out_type=jax.ShapeDtypeStruct((NW * 128,), jnp.float32),
    mesh=_MESH,
    scratch_types=[
        pltpu.VMEM((NZ,), jnp.float32),
        pltpu.VMEM((B12 // 128, 128), jnp.int32),
        pltpu.VMEM((B12 // 128, 128), jnp.int32),
        pltpu.VMEM((B12,), jnp.float32),
        pltpu.VMEM((128,), jnp.float32),
    ],
    compiler_params=_SC_PARAMS,
)


def _sc_pass2_body(src2_hbm, dst2_hbm, ew_hbm, z_hbm, ab_hbm,
                   wn_hbm,
                   z_v, src2_v, dst2_v, ew_v, wn_v, ab_v):
  c = lax.axis_index("c")
  s = lax.axis_index("s")
  wid = s * NC + c
  pltpu.sync_copy(z_hbm, z_v)
  pltpu.sync_copy(ab_hbm, ab_v)

  av = ab_v[pl.ds(0, L)]
  bv = ab_v[pl.ds(L, L)]
  base = wid * EW_W

  def batch_body(b, carry):
    off = base + b * B2
    pltpu.sync_copy(src2_hbm.at[pl.ds(off // 128, B2 // 128)], src2_v)
    pltpu.sync_copy(dst2_hbm.at[pl.ds(off // 128, B2 // 128)], dst2_v)
    pltpu.sync_copy(ew_hbm.at[pl.ds(off, B2)], ew_v)

    @plsc.parallel_loop(0, NG2, unroll=2)
    def _wn_group(g):
      i0 = g * L
      sv = _read16(src2_v, g)
      dv = _read16(dst2_v, g)
      zj = plsc.load_gather(z_v, [sv])
      zi = plsc.load_gather(z_v, [dv])
      ewv = ew_v[pl.ds(i0, L)]
      dd = ewv - 0.5 * (zi + zj)
      ee = jnp.exp(-dd)
      wp = ee * ee - 2.0 * ee
      wn_v[pl.ds(i0, L)] = av * wp + bv
    pltpu.sync_copy(wn_v, wn_hbm.at[pl.ds(off, B2)])
    return carry

  lax.fori_loop(0, NB2, batch_body, 0)


_sc_pass2 = pl.kernel(
    _sc_pass2_body,
    out_type=jax.ShapeDtypeStruct((E_PAD,), jnp.float32),
    mesh=_MESH,
    scratch_types=[
        pltpu.VMEM((NZ,), jnp.float32),
        pltpu.VMEM((B2 // 128, 128), jnp.int32),
        pltpu.VMEM((B2 // 128, 128), jnp.int32),
        pltpu.VMEM((B2,), jnp.float32),
        pltpu.VMEM((B2,), jnp.float32),
        pltpu.VMEM((2 * L,), jnp.float32),
    ],
    compiler_params=_SC_PARAMS,
)


def _sc_spmm_body(src2_hbm, dst2_hbm, wn_hbm, ew_hbm, ea_hbm, tab_hbm,
                  zz32_hbm, zzv_hbm,
                  g_hbm, segp_hbm,
                  src2_v, dst2_v, wn_v, ew_v, ea_v, rows_v, val16_v, st_v,
                  src2b_v, dst2b_v, wnb_v, rowsb_v,
                  acc, sem, sem_gb, sem_sa, sem_sb):
  c = lax.axis_index("c")
  s = lax.axis_index("s")
  base = s * EW_T
  pltpu.sync_copy(zz32_hbm, st_v)

  def _scale_rows(rows_ref, wn_ref):
    @plsc.parallel_loop(0, NGS, unroll=4)
    def _scale(g):
      i0 = g * L
      wv = wn_ref[pl.ds(i0, L)]
      ridx = _iota16() + i0
      for col in range(CW):
        cidx = jnp.full((L,), col, jnp.int32)
        v = plsc.load_gather(rows_ref, [ridx, cidx]) * wv
        plsc.store_scatter(rows_ref, [ridx, cidx], v)

  for r in range(NR):
    grp = c * NR + r
    for k in range(3):
      pltpu.sync_copy(st_v, acc.at[pl.ds(s * RPTS + k * CHS, CHS)])
    plsc.subcore_barrier()

    def pair_body(bb, carry):
      off_a = base + (2 * bb) * BS
      off_b = off_a + BS
      pltpu.sync_copy(src2_hbm.at[pl.ds(off_a // 128, BS // 128)], src2_v)
      pltpu.sync_copy(dst2_hbm.at[pl.ds(off_a // 128, BS // 128)], dst2_v)
      pltpu.sync_copy(wn_hbm.at[pl.ds(off_a, BS)], wn_v)
      pltpu.sync_copy(src2_hbm.at[pl.ds(off_b // 128, BS // 128)], src2b_v)
      pltpu.sync_copy(dst2_hbm.at[pl.ds(off_b // 128, BS // 128)], dst2b_v)
      pltpu.sync_copy(wn_hbm.at[pl.ds(off_b, BS)], wnb_v)
      gds_a = [
          pltpu.async_copy(tab_hbm.at[grp].at[src2_v.at[j]],
                           rows_v.at[pl.ds(j * 128, 128)], sem)
          for j in range(BS // 128)
      ]
      gds_b = [
          pltpu.async_copy(tab_hbm.at[grp].at[src2b_v.at[j]],
                           rowsb_v.at[pl.ds(j * 128, 128)], sem_gb)
          for j in range(BS // 128)
      ]
      for d in gds_a:
        d.wait()
      _scale_rows(rows_v, wn_v)
      sds_a = [
          pltpu.async_copy(rows_v.at[pl.ds(j * 128, 128)],
                           acc.at[dst2_v.at[j]], sem_sa, add=True)
          for j in range(BS // 128)
      ]
      for d in gds_b:
        d.wait()
      _scale_rows(rowsb_v, wnb_v)
      for d in sds_a:
        d.wait()
      sds_b = [
          pltpu.async_copy(rowsb_v.at[pl.ds(j * 128, 128)],
                           acc.at[dst2b_v.at[j]], sem_sb, add=True)
          for j in range(BS // 128)
      ]
      for d in sds_b:
        d.wait()
      return carry

    lax.fori_loop(0, NBS // 2, pair_body, 0)
    plsc.subcore_barrier()
    for k in range(3):
      r0 = s * RPTS + k * CHS
      pltpu.sync_copy(acc.at[pl.ds(r0, CHS)], st_v)
      pltpu.sync_copy(st_v, g_hbm.at[grp].at[pl.ds(r0, CHS)])
    plsc.subcore_barrier()
    pltpu.sync_copy(zz32_hbm, st_v)

  pltpu.sync_copy(zzv_hbm, val16_v)
  for k in range(3):
    pltpu.sync_copy(st_v, acc.at[pl.ds(s * RPTS + k * CHS, CHS)])
  plsc.subcore_barrier()
  ones = jnp.full((L,), 1.0, jnp.float32)
  sbase = c * (E_PAD // 2) + s * (E_PAD // 2 // NS)

  def seg_batch(b, carry):
    off = sbase + b * BS
    pltpu.sync_copy(dst2_hbm.at[pl.ds(off // 128, BS // 128)], dst2_v)
    pltpu.sync_copy(wn_hbm.at[pl.ds(off, BS)], wn_v)
    pltpu.sync_copy(ew_hbm.at[pl.ds(off, BS)], ew_v)
    pltpu.sync_copy(ea_hbm.at[pl.ds(off, BS)], ea_v)

    @plsc.parallel_loop(0, NGS, unroll=4)
    def _seg_group(g):
      i0 = g * L
      wn = wn_v[pl.ds(i0, L)]
      ewv = ew_v[pl.ds(i0, L)]
      eav = ea_v[pl.ds(i0, L)]
      ridx = _iota16() + i0
      plsc.store_scatter(val16_v, [ridx, jnp.zeros((L,), jnp.int32)], wn)
      plsc.store_scatter(val16_v, [ridx, jnp.full((L,), 1, jnp.int32)],
                         ewv * wn)
      plsc.store_scatter(val16_v, [ridx, jnp.full((L,), 2, jnp.int32)],
                         eav * wn)
      plsc.store_scatter(val16_v, [ridx, jnp.full((L,), 3, jnp.int32)], ones)
    descs = [
        pltpu.async_copy(val16_v.at[pl.ds(j * 128, 128)],
                         acc.at[dst2_v.at[j]], sem, add=True)
        for j in range(BS // 128)
    ]
    for d in descs:
      d.wait()
    return carry

  lax.fori_loop(0, E_PAD // 2 // NS // BS, seg_batch, 0)
  plsc.subcore_barrier()
  for k in range(3):
    r0 = s * RPTS + k * CHS
    pltpu.sync_copy(acc.at[pl.ds(r0, CHS)], st_v)
    pltpu.sync_copy(st_v, segp_hbm.at[c].at[pl.ds(r0, CHS)])


_sc_spmm = pl.kernel(
    _sc_spmm_body,
    out_type=[
        jax.ShapeDtypeStruct((NCG, NPS, CW), jnp.float32),
        jax.ShapeDtypeStruct((NC, NPS, CW), jnp.float32),
    ],
    mesh=_MESH,
    scratch_types=[
        pltpu.VMEM((BS // 128, 128), jnp.int32),
        pltpu.VMEM((BS // 128, 128), jnp.int32),
        pltpu.VMEM((BS,), jnp.float32),
        pltpu.VMEM((BS,), jnp.float32),
        pltpu.VMEM((BS,), jnp.float32),
        pltpu.VMEM((BS, CW), jnp.float32),
        pltpu.VMEM((BS, CW), jnp.float32),
        pltpu.VMEM((CHS, CW), jnp.float32),
        pltpu.VMEM((BS // 128, 128), jnp.int32),
        pltpu.VMEM((BS // 128, 128), jnp.int32),
        pltpu.VMEM((BS,), jnp.float32),
        pltpu.VMEM((BS, CW), jnp.float32),
        pltpu.VMEM_SHARED((NPS, CW), jnp.float32),
        pltpu.SemaphoreType.DMA,
        pltpu.SemaphoreType.DMA,
        pltpu.SemaphoreType.DMA,
        pltpu.SemaphoreType.DMA,
    ],
    compiler_params=_SC_PARAMS,
)


_RB = 2000
_GRID = N // _RB


def _tc0_body(x_ref, w0t_ref, b0_ref, hs_ref):
  v = jnp.dot(x_ref[...], w0t_ref[...],
              preferred_element_type=jnp.float32) + b0_ref[...]
  h = jnp.where(v >= 0.0, v, 0.01 * v)
  for k in range(NCG):
    hs_ref[k] = h[:, k * CW:(k + 1) * CW]


def _tc0(x, w0t, b0):
  return pl.pallas_call(
      _tc0_body,
      grid=(_GRID,),
      in_specs=[
          pl.BlockSpec((_RB, FIN), lambda i: (i, 0)),
          pl.BlockSpec((FIN, D), lambda i: (0, 0)),
          pl.BlockSpec((1, D), lambda i: (0, 0)),
      ],
      out_specs=pl.BlockSpec((NCG, _RB, CW), lambda i: (0, i, 0)),
      out_shape=jax.ShapeDtypeStruct((NCG, N, CW), jnp.float32),
  )(x, w0t, b0)


def _tc_dense_body(g_ref, hs_ref, seg_ref,
                   wat_ref, wbt_ref, wew_ref, wea_ref,
                   l2t_ref, b2_ref, l3t_ref, b3_ref,
                   wih_ref, bih_ref, whh_ref, bhh_ref,
                   o_ref):
  seg = seg_ref[0][:, :4] + seg_ref[1][:, :4]
  s_wn = seg[:, 0:1]
  s_ew = seg[:, 1:2]
  s_ea = seg[:, 2:3]
  cnt = seg[:, 3:4]
  g = jnp.concatenate([g_ref[k] for k in range(NCG)], axis=1)
  h = jnp.concatenate([hs_ref[k] for k in range(NCG)], axis=1)

  dot = functools.partial(jnp.dot, preferred_element_type=jnp.float32)
  sums = (dot(h * s_wn, wat_ref[...]) + dot(g, wbt_ref[...])
          + s_ew * wew_ref[...] + s_ea * wea_ref[...])
  agg = sums / jnp.maximum(cnt, 1.0)
  m = dot(agg, l2t_ref[...]) + b2_ref[...]
  m = jnp.maximum(m, 0.0) + jnp.log1p(jnp.exp(-jnp.abs(m))) - 0.6931471805599453
  m = dot(m, l3t_ref[...]) + b3_ref[...]
  gi = dot(m, wih_ref[...]) + bih_ref[...]
  gh = dot(h, whh_ref[...]) + bhh_ref[...]
  r = jax.nn.sigmoid(gi[:, :D] + gh[:, :D])
  zt = jax.nn.sigmoid(gi[:, D:2 * D] + gh[:, D:2 * D])
  ng = jnp.tanh(gi[:, 2 * D:] + r * gh[:, 2 * D:])
  hn = (1.0 - zt) * ng + zt * h
  for k in range(NCG):
    o_ref[k] = hn[:, k * CW:(k + 1) * CW]


def _tc_dense(g2, hs, segp, wat, wbt, wew, wea, l2t, b2, l3t, b3,
              wih, bih, whh, bhh):
  full = lambda shape: pl.BlockSpec(shape, lambda i, _s=shape: tuple(0 for _ in _s))
  return pl.pallas_call(
      _tc_dense_body,
      grid=(_GRID,),
      in_specs=[
          pl.BlockSpec((NCG, _RB, CW), lambda i: (0, i, 0)),
          pl.BlockSpec((NCG, _RB, CW), lambda i: (0, i, 0)),
          pl.BlockSpec((NC, _RB, CW), lambda i: (0, i, 0)),
          full((D, 2 * D)), full((D, 2 * D)), full((1, 2 * D)),
          full((1, 2 * D)),
          full((2 * D, 2 * D)), full((1, 2 * D)), full((2 * D, D)),
          full((1, D)),
          full((D, 3 * D)), full((1, 3 * D)), full((D, 3 * D)),
          full((1, 3 * D)),
      ],
      out_specs=pl.BlockSpec((NCG, _RB, CW), lambda i: (0, i, 0)),
      out_shape=jax.ShapeDtypeStruct((NCG, N, CW), jnp.float32),
  )(g2, hs, segp, wat, wbt, wew, wea, l2t, b2, l3t, b3, wih, bih, whh, bhh)


def kernel(x, edge_index, edge_weight, edge_attr, z, W0, b0, lin1_W,
           lin2_W, lin2_b, lin3_W, lin3_b, bn_gamma, bn_beta,
           gru_Wih, gru_Whh, gru_bih, gru_bhh):
  f32 = jnp.float32
  src = edge_index[0]
  dst = edge_index[1]
  npad = E_PAD - E
  srcp = jnp.concatenate([src, jnp.zeros((npad,), jnp.int32)])
  dstp = jnp.concatenate([dst, jnp.full((npad,), N, jnp.int32)])
  srcp2 = srcp.reshape(E_PAD // 128, 128)
  dstp2 = dstp.reshape(E_PAD // 128, 128)
  ewp = jnp.concatenate([edge_weight, jnp.full((npad,), 20.0, f32)])
  eap = jnp.concatenate([edge_attr[:, 0], jnp.zeros((npad,), f32)])
  zpad = jnp.concatenate([z[:, 0], jnp.zeros((NZ - N,), f32)])

  w0t = W0.T
  b0r = b0.reshape(1, D)
  wat = lin1_W[:, :D].T
  wbt = lin1_W[:, D:2 * D].T
  wew = lin1_W[:, 2 * D].reshape(1, 2 * D)
  wea = lin1_W[:, 2 * D + 1].reshape(1, 2 * D)
  l2t = lin2_W.T
  b2r = lin2_b.reshape(1, 2 * D)
  l3t = lin3_W.T
  b3r = lin3_b.reshape(1, D)
  wih = gru_Wih.T
  bih = gru_bih.reshape(1, 3 * D)
  whh = gru_Whh.T
  bhh = gru_bhh.reshape(1, 3 * D)

  zzv = jnp.zeros((BS, CW), f32)
  zz32 = jnp.zeros((CHS, CW), f32)

  part = _sc_pass1(srcp2, dstp2, ewp, zpad).reshape(NW, 128)
  s1 = jnp.sum(part[:, :L])
  s2 = jnp.sum(part[:, L:2 * L])
  mu = s1 / E
  var = s2 / E - mu * mu
  a = bn_gamma[0] / jnp.sqrt(var + 1e-5)
  b_ = bn_beta[0] - mu * a
  ab = jnp.concatenate([jnp.full((L,), a, f32), jnp.full((L,), b_, f32)])

  wn = _sc_pass2(srcp2, dstp2, ewp, zpad, ab)

  hs = _tc0(x, w0t, b0r)

  for _ in range(3):
    g2, segp = _sc_spmm(srcp2, dstp2, wn, ewp, eap, hs, zz32, zzv)
    hs = _tc_dense(g2, hs, segp, wat, wbt, wew, wea,
                   l2t, b2r, l3t, b3r, wih, bih, whh, bhh)

  return jnp.concatenate([hs[k] for k in range(NCG)], axis=1)


# speedup vs baseline: 7.5507x; 1.1331x over previous
import functools
import jax
import jax.numpy as jnp
from jax import lax
from jax.experimental import pallas as pl
from jax.experimental.pallas import tpu as pltpu
from jax.experimental.pallas import tpu_sc as plsc

N = 50000
E = 800000
FIN = 19
D = 64

NC = 2
NS = 16
NW = NC * NS
L = 16

E_PAD = 819200
NZ = N + 16
NPS = 50016

EW_W = E_PAD // NW
B12 = 5120
NB12 = EW_W // B12
NG12 = B12 // L

EW_T = E_PAD // NS
BS = 1024
NBS = EW_T // BS
NGS = BS // L

RPTS = NPS // NS
CHS = RPTS // 3

B2 = 1280
NB2 = EW_W // B2
NG2 = B2 // L

CW = 16
NCG = D // CW
NR = NCG // NC

_MESH = plsc.VectorSubcoreMesh(
    core_axis_name="c", subcore_axis_name="s", num_cores=NC, num_subcores=NS)

_SC_PARAMS = pltpu.CompilerParams(
    needs_layout_passes=False, use_tc_tiling_on_sc=False)


def _iota16():
  return lax.iota(jnp.int32, L)


def _read16(ref2d, g):
  row = jnp.full((L,), g // 8, jnp.int32)
  col = _iota16() + (g % 8) * L
  return plsc.load_gather(ref2d, [row, col])


def _sc_pass1_body(src2_hbm, dst2_hbm, ew_hbm, z_hbm, out_hbm,
                   z_v, src2_v, dst2_v, ew_v, res_v):
  c = lax.axis_index("c")
  s = lax.axis_index("s")
  wid = s * NC + c
  pltpu.sync_copy(z_hbm, z_v)
  base = wid * EW_W

  def batch_body(b, carry):
    s1, s2 = carry
    off = base + b * B12
    pltpu.sync_copy(src2_hbm.at[pl.ds(off // 128, B12 // 128)], src2_v)
    pltpu.sync_copy(dst2_hbm.at[pl.ds(off // 128, B12 // 128)], dst2_v)
    pltpu.sync_copy(ew_hbm.at[pl.ds(off, B12)], ew_v)

    def group_body(g, carry2):
      t1, t2 = carry2
      i0 = g * L
      sv = _read16(src2_v, g)
      dv = _read16(dst2_v, g)
      zj = plsc.load_gather(z_v, [sv])
      zi = plsc.load_gather(z_v, [dv])
      ewv = ew_v[pl.ds(i0, L)]
      dd = ewv - 0.5 * (zi + zj)
      ee = jnp.exp(-dd)
      wp = ee * ee - 2.0 * ee
      return (t1 + wp, t2 + wp * wp)

    return lax.fori_loop(0, NG12, group_body, (s1, s2))

  z16 = jnp.zeros((L,), jnp.float32)
  s1, s2 = lax.fori_loop(0, NB12, batch_body, (z16, z16))
  res_v[pl.ds(0, L)] = s1
  res_v[pl.ds(L, L)] = s2
  pltpu.sync_copy(res_v, out_hbm.at[pl.ds(wid * 128, 128)])


_sc_pass1 = pl.kernel(
    _sc_pass1_body,
    out_type=jax.ShapeDtypeStruct((NW * 128,), jnp.float32),
    mesh=_MESH,
    scratch_types=[
        pltpu.VMEM((NZ,), jnp.float32),
        pltpu.VMEM((B12 // 128, 128), jnp.int32),
        pltpu.VMEM((B12 // 128, 128), jnp.int32),
        pltpu.VMEM((B12,), jnp.float32),
        pltpu.VMEM((128,), jnp.float32),
    ],
    compiler_params=_SC_PARAMS,
)


def _sc_pass2_body(src2_hbm, dst2_hbm, ew_hbm, z_hbm, ab_hbm,
                   wn_hbm,
                   z_v, src2_v, dst2_v, ew_v, wn_v, ab_v):
  c = lax.axis_index("c")
  s = lax.axis_index("s")
  wid = s * NC + c
  pltpu.sync_copy(z_hbm, z_v)
  pltpu.sync_copy(ab_hbm, ab_v)

  av = ab_v[pl.ds(0, L)]
  bv = ab_v[pl.ds(L, L)]
  base = wid * EW_W

  def batch_body(b, carry):
    off = base + b * B2
    pltpu.sync_copy(src2_hbm.at[pl.ds(off // 128, B2 // 128)], src2_v)
    pltpu.sync_copy(dst2_hbm.at[pl.ds(off // 128, B2 // 128)], dst2_v)
    pltpu.sync_copy(ew_hbm.at[pl.ds(off, B2)], ew_v)

    @plsc.parallel_loop(0, NG2, unroll=2)
    def _wn_group(g):
      i0 = g * L
      sv = _read16(src2_v, g)
      dv = _read16(dst2_v, g)
      zj = plsc.load_gather(z_v, [sv])
      zi = plsc.load_gather(z_v, [dv])
      ewv = ew_v[pl.ds(i0, L)]
      dd = ewv - 0.5 * (zi + zj)
      ee = jnp.exp(-dd)
      wp = ee * ee - 2.0 * ee
      wn_v[pl.ds(i0, L)] = av * wp + bv
    pltpu.sync_copy(wn_v, wn_hbm.at[pl.ds(off, B2)])
    return carry

  lax.fori_loop(0, NB2, batch_body, 0)


_sc_pass2 = pl.kernel(
    _sc_pass2_body,
    out_type=jax.ShapeDtypeStruct((E_PAD,), jnp.float32),
    mesh=_MESH,
    scratch_types=[
        pltpu.VMEM((NZ,), jnp.float32),
        pltpu.VMEM((B2 // 128, 128), jnp.int32),
        pltpu.VMEM((B2 // 128, 128), jnp.int32),
        pltpu.VMEM((B2,), jnp.float32),
        pltpu.VMEM((B2,), jnp.float32),
        pltpu.VMEM((2 * L,), jnp.float32),
    ],
    compiler_params=_SC_PARAMS,
)


def _sc_spmm_body(src2_hbm, dst2_hbm, wn_hbm, ew_hbm, ea_hbm, tab_hbm,
                  zz32_hbm, zzv_hbm,
                  g_hbm, segp_hbm,
                  src2_v, dst2_v, wn_v, ew_v, ea_v, rows_v, val16_v, st_v,
                  src2b_v, dst2b_v, wnb_v, rowsb_v,
                  acc, sem, sem_gb, sem_sa, sem_sb):
  c = lax.axis_index("c")
  s = lax.axis_index("s")
  base = s * EW_T
  pltpu.sync_copy(zz32_hbm, st_v)

  def _scale_rows(rows_ref, wn_ref):
    @plsc.parallel_loop(0, NGS, unroll=4)
    def _scale(g):
      i0 = g * L
      wv = wn_ref[pl.ds(i0, L)]
      ridx = _iota16() + i0
      for col in range(CW):
        cidx = jnp.full((L,), col, jnp.int32)
        v = plsc.load_gather(rows_ref, [ridx, cidx]) * wv
        plsc.store_scatter(rows_ref, [ridx, cidx], v)

  for r in range(NR):
    grp = c * NR + r
    for k in range(3):
      pltpu.sync_copy(st_v, acc.at[pl.ds(s * RPTS + k * CHS, CHS)])
    plsc.subcore_barrier()

    def pair_body(bb, carry):
      off_a = base + (2 * bb) * BS
      off_b = off_a + BS
      pltpu.sync_copy(src2_hbm.at[pl.ds(off_a // 128, BS // 128)], src2_v)
      gds_a = [
          pltpu.async_copy(tab_hbm.at[grp].at[src2_v.at[j]],
                           rows_v.at[pl.ds(j * 128, 128)], sem)
          for j in range(BS // 128)
      ]
      pltpu.sync_copy(dst2_hbm.at[pl.ds(off_a // 128, BS // 128)], dst2_v)
      pltpu.sync_copy(wn_hbm.at[pl.ds(off_a, BS)], wn_v)
      pltpu.sync_copy(src2_hbm.at[pl.ds(off_b // 128, BS // 128)], src2b_v)
      gds_b = [
          pltpu.async_copy(tab_hbm.at[grp].at[src2b_v.at[j]],
                           rowsb_v.at[pl.ds(j * 128, 128)], sem_gb)
          for j in range(BS // 128)
      ]
      pltpu.sync_copy(dst2_hbm.at[pl.ds(off_b // 128, BS // 128)], dst2b_v)
      pltpu.sync_copy(wn_hbm.at[pl.ds(off_b, BS)], wnb_v)
      for d in gds_a:
        d.wait()
      _scale_rows(rows_v, wn_v)
      sds_a = [
          pltpu.async_copy(rows_v.at[pl.ds(j * 128, 128)],
                           acc.at[dst2_v.at[j]], sem_sa, add=True)
          for j in range(BS // 128)
      ]
      for d in gds_b:
        d.wait()
      _scale_rows(rowsb_v, wnb_v)
      for d in sds_a:
        d.wait()
      sds_b = [
          pltpu.async_copy(rowsb_v.at[pl.ds(j * 128, 128)],
                           acc.at[dst2b_v.at[j]], sem_sb, add=True)
          for j in range(BS // 128)
      ]
      for d in sds_b:
        d.wait()
      return carry

    lax.fori_loop(0, NBS // 2, pair_body, 0)
    plsc.subcore_barrier()
    for k in range(3):
      r0 = s * RPTS + k * CHS
      pltpu.sync_copy(acc.at[pl.ds(r0, CHS)], st_v)
      pltpu.sync_copy(st_v, g_hbm.at[grp].at[pl.ds(r0, CHS)])
    plsc.subcore_barrier()
    pltpu.sync_copy(zz32_hbm, st_v)

  pltpu.sync_copy(zzv_hbm, val16_v)
  for k in range(3):
    pltpu.sync_copy(st_v, acc.at[pl.ds(s * RPTS + k * CHS, CHS)])
  plsc.subcore_barrier()
  ones = jnp.full((L,), 1.0, jnp.float32)
  sbase = c * (E_PAD // 2) + s * (E_PAD // 2 // NS)

  def seg_batch(b, carry):
    off = sbase + b * BS
    pltpu.sync_copy(dst2_hbm.at[pl.ds(off // 128, BS // 128)], dst2_v)
    pltpu.sync_copy(wn_hbm.at[pl.ds(off, BS)], wn_v)
    pltpu.sync_copy(ew_hbm.at[pl.ds(off, BS)], ew_v)
    pltpu.sync_copy(ea_hbm.at[pl.ds(off, BS)], ea_v)

    @plsc.parallel_loop(0, NGS, unroll=4)
    def _seg_group(g):
      i0 = g * L
      wn = wn_v[pl.ds(i0, L)]
      ewv = ew_v[pl.ds(i0, L)]
      eav = ea_v[pl.ds(i0, L)]
      ridx = _iota16() + i0
      plsc.store_scatter(val16_v, [ridx, jnp.zeros((L,), jnp.int32)], wn)
      plsc.store_scatter(val16_v, [ridx, jnp.full((L,), 1, jnp.int32)],
                         ewv * wn)
      plsc.store_scatter(val16_v, [ridx, jnp.full((L,), 2, jnp.int32)],
                         eav * wn)
      plsc.store_scatter(val16_v, [ridx, jnp.full((L,), 3, jnp.int32)], ones)
    descs = [
        pltpu.async_copy(val16_v.at[pl.ds(j * 128, 128)],
                         acc.at[dst2_v.at[j]], sem, add=True)
        for j in range(BS // 128)
    ]
    for d in descs:
      d.wait()
    return carry

  lax.fori_loop(0, E_PAD // 2 // NS // BS, seg_batch, 0)
  plsc.subcore_barrier()
  for k in range(3):
    r0 = s * RPTS + k * CHS
    pltpu.sync_copy(acc.at[pl.ds(r0, CHS)], st_v)
    pltpu.sync_copy(st_v, segp_hbm.at[c].at[pl.ds(r0, CHS)])


_sc_spmm = pl.kernel(
    _sc_spmm_body,
    out_type=[
        jax.ShapeDtypeStruct((NCG, NPS, CW), jnp.float32),
        jax.ShapeDtypeStruct((NC, NPS, CW), jnp.float32),
    ],
    mesh=_MESH,
    scratch_types=[
        pltpu.VMEM((BS // 128, 128), jnp.int32),
        pltpu.VMEM((BS // 128, 128), jnp.int32),
        pltpu.VMEM((BS,), jnp.float32),
        pltpu.VMEM((BS,), jnp.float32),
        pltpu.VMEM((BS,), jnp.float32),
        pltpu.VMEM((BS, CW), jnp.float32),
        pltpu.VMEM((BS, CW), jnp.float32),
        pltpu.VMEM((CHS, CW), jnp.float32),
        pltpu.VMEM((BS // 128, 128), jnp.int32),
        pltpu.VMEM((BS // 128, 128), jnp.int32),
        pltpu.VMEM((BS,), jnp.float32),
        pltpu.VMEM((BS, CW), jnp.float32),
        pltpu.VMEM_SHARED((NPS, CW), jnp.float32),
        pltpu.SemaphoreType.DMA,
        pltpu.SemaphoreType.DMA,
        pltpu.SemaphoreType.DMA,
        pltpu.SemaphoreType.DMA,
    ],
    compiler_params=_SC_PARAMS,
)


_RB = 2000
_GRID = N // _RB


def _tc0_body(x_ref, w0t_ref, b0_ref, hs_ref):
  v = jnp.dot(x_ref[...], w0t_ref[...],
              preferred_element_type=jnp.float32) + b0_ref[...]
  h = jnp.where(v >= 0.0, v, 0.01 * v)
  for k in range(NCG):
    hs_ref[k] = h[:, k * CW:(k + 1) * CW]


def _tc0(x, w0t, b0):
  return pl.pallas_call(
      _tc0_body,
      grid=(_GRID,),
      in_specs=[
          pl.BlockSpec((_RB, FIN), lambda i: (i, 0)),
          pl.BlockSpec((FIN, D), lambda i: (0, 0)),
          pl.BlockSpec((1, D), lambda i: (0, 0)),
      ],
      out_specs=pl.BlockSpec((NCG, _RB, CW), lambda i: (0, i, 0)),
      out_shape=jax.ShapeDtypeStruct((NCG, N, CW), jnp.float32),
  )(x, w0t, b0)


def _tc_dense_body(g_ref, hs_ref, seg_ref,
                   wat_ref, wbt_ref, wew_ref, wea_ref,
                   l2t_ref, b2_ref, l3t_ref, b3_ref,
                   wih_ref, bih_ref, whh_ref, bhh_ref,
                   o_ref):
  seg = seg_ref[0][:, :4] + seg_ref[1][:, :4]
  s_wn = seg[:, 0:1]
  s_ew = seg[:, 1:2]
  s_ea = seg[:, 2:3]
  cnt = seg[:, 3:4]
  g = jnp.concatenate([g_ref[k] for k in range(NCG)], axis=1)
  h = jnp.concatenate([hs_ref[k] for k in range(NCG)], axis=1)

  dot = functools.partial(jnp.dot, preferred_element_type=jnp.float32)
  sums = (dot(h * s_wn, wat_ref[...]) + dot(g, wbt_ref[...])
          + s_ew * wew_ref[...] + s_ea * wea_ref[...])
  agg = sums / jnp.maximum(cnt, 1.0)
  m = dot(agg, l2t_ref[...]) + b2_ref[...]
  m = jnp.maximum(m, 0.0) + jnp.log1p(jnp.exp(-jnp.abs(m))) - 0.6931471805599453
  m = dot(m, l3t_ref[...]) + b3_ref[...]
  gi = dot(m, wih_ref[...]) + bih_ref[...]
  gh = dot(h, whh_ref[...]) + bhh_ref[...]
  r = jax.nn.sigmoid(gi[:, :D] + gh[:, :D])
  zt = jax.nn.sigmoid(gi[:, D:2 * D] + gh[:, D:2 * D])
  ng = jnp.tanh(gi[:, 2 * D:] + r * gh[:, 2 * D:])
  hn = (1.0 - zt) * ng + zt * h
  for k in range(NCG):
    o_ref[k] = hn[:, k * CW:(k + 1) * CW]


def _tc_dense(g2, hs, segp, wat, wbt, wew, wea, l2t, b2, l3t, b3,
              wih, bih, whh, bhh):
  full = lambda shape: pl.BlockSpec(shape, lambda i, _s=shape: tuple(0 for _ in _s))
  return pl.pallas_call(
      _tc_dense_body,
      grid=(_GRID,),
      in_specs=[
          pl.BlockSpec((NCG, _RB, CW), lambda i: (0, i, 0)),
          pl.BlockSpec((NCG, _RB, CW), lambda i: (0, i, 0)),
          pl.BlockSpec((NC, _RB, CW), lambda i: (0, i, 0)),
          full((D, 2 * D)), full((D, 2 * D)), full((1, 2 * D)),
          full((1, 2 * D)),
          full((2 * D, 2 * D)), full((1, 2 * D)), full((2 * D, D)),
          full((1, D)),
          full((D, 3 * D)), full((1, 3 * D)), full((D, 3 * D)),
          full((1, 3 * D)),
      ],
      out_specs=pl.BlockSpec((NCG, _RB, CW), lambda i: (0, i, 0)),
      out_shape=jax.ShapeDtypeStruct((NCG, N, CW), jnp.float32),
  )(g2, hs, segp, wat, wbt, wew, wea, l2t, b2, l3t, b3, wih, bih, whh, bhh)


def kernel(x, edge_index, edge_weight, edge_attr, z, W0, b0, lin1_W,
           lin2_W, lin2_b, lin3_W, lin3_b, bn_gamma, bn_beta,
           gru_Wih, gru_Whh, gru_bih, gru_bhh):
  f32 = jnp.float32
  src = edge_index[0]
  dst = edge_index[1]
  npad = E_PAD - E
  srcp = jnp.concatenate([src, jnp.zeros((npad,), jnp.int32)])
  dstp = jnp.concatenate([dst, jnp.full((npad,), N, jnp.int32)])
  srcp2 = srcp.reshape(E_PAD // 128, 128)
  dstp2 = dstp.reshape(E_PAD // 128, 128)
  ewp = jnp.concatenate([edge_weight, jnp.full((npad,), 20.0, f32)])
  eap = jnp.concatenate([edge_attr[:, 0], jnp.zeros((npad,), f32)])
  zpad = jnp.concatenate([z[:, 0], jnp.zeros((NZ - N,), f32)])

  w0t = W0.T
  b0r = b0.reshape(1, D)
  wat = lin1_W[:, :D].T
  wbt = lin1_W[:, D:2 * D].T
  wew = lin1_W[:, 2 * D].reshape(1, 2 * D)
  wea = lin1_W[:, 2 * D + 1].reshape(1, 2 * D)
  l2t = lin2_W.T
  b2r = lin2_b.reshape(1, 2 * D)
  l3t = lin3_W.T
  b3r = lin3_b.reshape(1, D)
  wih = gru_Wih.T
  bih = gru_bih.reshape(1, 3 * D)
  whh = gru_Whh.T
  bhh = gru_bhh.reshape(1, 3 * D)

  zzv = jnp.zeros((BS, CW), f32)
  zz32 = jnp.zeros((CHS, CW), f32)

  part = _sc_pass1(srcp2, dstp2, ewp, zpad).reshape(NW, 128)
  s1 = jnp.sum(part[:, :L])
  s2 = jnp.sum(part[:, L:2 * L])
  mu = s1 / E
  var = s2 / E - mu * mu
  a = bn_gamma[0] / jnp.sqrt(var + 1e-5)
  b_ = bn_beta[0] - mu * a
  ab = jnp.concatenate([jnp.full((L,), a, f32), jnp.full((L,), b_, f32)])

  wn = _sc_pass2(srcp2, dstp2, ewp, zpad, ab)

  hs = _tc0(x, w0t, b0r)

  for _ in range(3):
    g2, segp = _sc_spmm(srcp2, dstp2, wn, ewp, eap, hs, zz32, zzv)
    hs = _tc_dense(g2, hs, segp, wat, wbt, wew, wea,
                   l2t, b2r, l3t, b3r, wih, bih, whh, bhh)

  return jnp.concatenate([hs[k] for k in range(NCG)], axis=1)


# speedup vs baseline: 7.6449x; 1.0125x over previous
import functools
import jax
import jax.numpy as jnp
from jax import lax
from jax.experimental import pallas as pl
from jax.experimental.pallas import tpu as pltpu
from jax.experimental.pallas import tpu_sc as plsc

N = 50000
E = 800000
FIN = 19
D = 64

NC = 2
NS = 16
NW = NC * NS
L = 16

E_PAD = 819200
NZ = N + 16
NPS = 50016

EW_W = E_PAD // NW
B12 = 5120
NB12 = EW_W // B12
NG12 = B12 // L

EW_T = E_PAD // NS
BS = 1024
NBS = EW_T // BS
NGS = BS // L

RPTS = NPS // NS
CHS = RPTS // 3

B2 = 1280
NB2 = EW_W // B2
NG2 = B2 // L

CW = 16
NCG = D // CW
NR = NCG // NC

_MESH = plsc.VectorSubcoreMesh(
    core_axis_name="c", subcore_axis_name="s", num_cores=NC, num_subcores=NS)

_SC_PARAMS = pltpu.CompilerParams(
    needs_layout_passes=False, use_tc_tiling_on_sc=False)


def _iota16():
  return lax.iota(jnp.int32, L)


def _read16(ref2d, g):
  row = jnp.full((L,), g // 8, jnp.int32)
  col = _iota16() + (g % 8) * L
  return plsc.load_gather(ref2d, [row, col])


def _sc_pass1_body(src2_hbm, dst2_hbm, ew_hbm, z_hbm, out_hbm,
                   z_v, src2_v, dst2_v, ew_v, res_v):
  c = lax.axis_index("c")
  s = lax.axis_index("s")
  wid = s * NC + c
  pltpu.sync_copy(z_hbm, z_v)
  base = wid * EW_W

  def batch_body(b, carry):
    s1, s2 = carry
    off = base + b * B12
    pltpu.sync_copy(src2_hbm.at[pl.ds(off // 128, B12 // 128)], src2_v)
    pltpu.sync_copy(dst2_hbm.at[pl.ds(off // 128, B12 // 128)], dst2_v)
    pltpu.sync_copy(ew_hbm.at[pl.ds(off, B12)], ew_v)

    def group_body(g, carry2):
      t1, t2 = carry2
      i0 = g * L
      sv = _read16(src2_v, g)
      dv = _read16(dst2_v, g)
      zj = plsc.load_gather(z_v, [sv])
      zi = plsc.load_gather(z_v, [dv])
      ewv = ew_v[pl.ds(i0, L)]
      dd = ewv - 0.5 * (zi + zj)
      ee = jnp.exp(-dd)
      wp = ee * ee - 2.0 * ee
      return (t1 + wp, t2 + wp * wp)

    return lax.fori_loop(0, NG12, group_body, (s1, s2))

  z16 = jnp.zeros((L,), jnp.float32)
  s1, s2 = lax.fori_loop(0, NB12, batch_body, (z16, z16))
  res_v[pl.ds(0, L)] = s1
  res_v[pl.ds(L, L)] = s2
  pltpu.sync_copy(res_v, out_hbm.at[pl.ds(wid * 128, 128)])


_sc_pass1 = pl.kernel(
    _sc_pass1_body,
    out_type=jax.ShapeDtypeStruct((NW * 128,), jnp.float32),
    mesh=_MESH,
    scratch_types=[
        pltpu.VMEM((NZ,), jnp.float32),
        pltpu.VMEM((B12 // 128, 128), jnp.int32),
        pltpu.VMEM((B12 // 128, 128), jnp.int32),
        pltpu.VMEM((B12,), jnp.float32),
        pltpu.VMEM((128,), jnp.float32),
    ],
    compiler_params=_SC_PARAMS,
)


def _sc_pass2_body(src2_hbm, dst2_hbm, ew_hbm, z_hbm, ab_hbm,
                   wn_hbm,
                   z_v, src2_v, dst2_v, ew_v, wn_v, ab_v):
  c = lax.axis_index("c")
  s = lax.axis_index("s")
  wid = s * NC + c
  pltpu.sync_copy(z_hbm, z_v)
  pltpu.sync_copy(ab_hbm, ab_v)

  av = ab_v[pl.ds(0, L)]
  bv = ab_v[pl.ds(L, L)]
  base = wid * EW_W

  def batch_body(b, carry):
    off = base + b * B2
    pltpu.sync_copy(src2_hbm.at[pl.ds(off // 128, B2 // 128)], src2_v)
    pltpu.sync_copy(dst2_hbm.at[pl.ds(off // 128, B2 // 128)], dst2_v)
    pltpu.sync_copy(ew_hbm.at[pl.ds(off, B2)], ew_v)

    @plsc.parallel_loop(0, NG2, unroll=2)
    def _wn_group(g):
      i0 = g * L
      sv = _read16(src2_v, g)
      dv = _read16(dst2_v, g)
      zj = plsc.load_gather(z_v, [sv])
      zi = plsc.load_gather(z_v, [dv])
      ewv = ew_v[pl.ds(i0, L)]
      dd = ewv - 0.5 * (zi + zj)
      ee = jnp.exp(-dd)
      wp = ee * ee - 2.0 * ee
      wn_v[pl.ds(i0, L)] = av * wp + bv
    pltpu.sync_copy(wn_v, wn_hbm.at[pl.ds(off, B2)])
    return carry

  lax.fori_loop(0, NB2, batch_body, 0)


_sc_pass2 = pl.kernel(
    _sc_pass2_body,
    out_type=jax.ShapeDtypeStruct((E_PAD,), jnp.float32),
    mesh=_MESH,
    scratch_types=[
        pltpu.VMEM((NZ,), jnp.float32),
        pltpu.VMEM((B2 // 128, 128), jnp.int32),
        pltpu.VMEM((B2 // 128, 128), jnp.int32),
        pltpu.VMEM((B2,), jnp.float32),
        pltpu.VMEM((B2,), jnp.float32),
        pltpu.VMEM((2 * L,), jnp.float32),
    ],
    compiler_params=_SC_PARAMS,
)


def _sc_spmm_body(src2_hbm, dst2_hbm, wn_hbm, ew_hbm, ea_hbm, tab_hbm,
                  zz32_hbm, zzv_hbm,
                  g_hbm, segp_hbm,
                  src2_v, dst2_v, wn_v, ew_v, ea_v, rows_v, val16_v, st_v,
                  src2b_v, dst2b_v, wnb_v, rowsb_v,
                  acc, sem, sem_gb, sem_sa, sem_sb,
                  sem_wa, sem_da, sem_wb, sem_db):
  c = lax.axis_index("c")
  s = lax.axis_index("s")
  base = s * EW_T
  pltpu.sync_copy(zz32_hbm, st_v)

  def _scale_rows(rows_ref, wn_ref):
    @plsc.parallel_loop(0, NGS, unroll=4)
    def _scale(g):
      i0 = g * L
      wv = wn_ref[pl.ds(i0, L)]
      ridx = _iota16() + i0
      for col in range(CW):
        cidx = jnp.full((L,), col, jnp.int32)
        v = plsc.load_gather(rows_ref, [ridx, cidx]) * wv
        plsc.store_scatter(rows_ref, [ridx, cidx], v)

  for r in range(NR):
    grp = c * NR + r
    for k in range(3):
      pltpu.sync_copy(st_v, acc.at[pl.ds(s * RPTS + k * CHS, CHS)])
    plsc.subcore_barrier()

    def pair_body(bb, carry):
      off_a = base + (2 * bb) * BS
      off_b = off_a + BS
      pltpu.sync_copy(src2_hbm.at[pl.ds(off_a // 128, BS // 128)], src2_v)
      gds_a = [
          pltpu.async_copy(tab_hbm.at[grp].at[src2_v.at[j]],
                           rows_v.at[pl.ds(j * 128, 128)], sem)
          for j in range(BS // 128)
      ]
      wd_a = pltpu.async_copy(wn_hbm.at[pl.ds(off_a, BS)], wn_v, sem_wa)
      dd_a = pltpu.async_copy(dst2_hbm.at[pl.ds(off_a // 128, BS // 128)],
                              dst2_v, sem_da)
      pltpu.sync_copy(src2_hbm.at[pl.ds(off_b // 128, BS // 128)], src2b_v)
      gds_b = [
          pltpu.async_copy(tab_hbm.at[grp].at[src2b_v.at[j]],
                           rowsb_v.at[pl.ds(j * 128, 128)], sem_gb)
          for j in range(BS // 128)
      ]
      wd_b = pltpu.async_copy(wn_hbm.at[pl.ds(off_b, BS)], wnb_v, sem_wb)
      dd_b = pltpu.async_copy(dst2_hbm.at[pl.ds(off_b // 128, BS // 128)],
                              dst2b_v, sem_db)
      for d in gds_a:
        d.wait()
      wd_a.wait()
      _scale_rows(rows_v, wn_v)
      dd_a.wait()
      sds_a = [
          pltpu.async_copy(rows_v.at[pl.ds(j * 128, 128)],
                           acc.at[dst2_v.at[j]], sem_sa, add=True)
          for j in range(BS // 128)
      ]
      for d in gds_b:
        d.wait()
      wd_b.wait()
      _scale_rows(rowsb_v, wnb_v)
      for d in sds_a:
        d.wait()
      dd_b.wait()
      sds_b = [
          pltpu.async_copy(rowsb_v.at[pl.ds(j * 128, 128)],
                           acc.at[dst2b_v.at[j]], sem_sb, add=True)
          for j in range(BS // 128)
      ]
      for d in sds_b:
        d.wait()
      return carry

    lax.fori_loop(0, NBS // 2, pair_body, 0)
    plsc.subcore_barrier()
    for k in range(3):
      r0 = s * RPTS + k * CHS
      pltpu.sync_copy(acc.at[pl.ds(r0, CHS)], st_v)
      pltpu.sync_copy(st_v, g_hbm.at[grp].at[pl.ds(r0, CHS)])
    plsc.subcore_barrier()
    pltpu.sync_copy(zz32_hbm, st_v)

  pltpu.sync_copy(zzv_hbm, val16_v)
  for k in range(3):
    pltpu.sync_copy(st_v, acc.at[pl.ds(s * RPTS + k * CHS, CHS)])
  plsc.subcore_barrier()
  ones = jnp.full((L,), 1.0, jnp.float32)
  sbase = c * (E_PAD // 2) + s * (E_PAD // 2 // NS)

  def seg_batch(b, carry):
    off = sbase + b * BS
    pltpu.sync_copy(dst2_hbm.at[pl.ds(off // 128, BS // 128)], dst2_v)
    pltpu.sync_copy(wn_hbm.at[pl.ds(off, BS)], wn_v)
    pltpu.sync_copy(ew_hbm.at[pl.ds(off, BS)], ew_v)
    pltpu.sync_copy(ea_hbm.at[pl.ds(off, BS)], ea_v)

    @plsc.parallel_loop(0, NGS, unroll=4)
    def _seg_group(g):
      i0 = g * L
      wn = wn_v[pl.ds(i0, L)]
      ewv = ew_v[pl.ds(i0, L)]
      eav = ea_v[pl.ds(i0, L)]
      ridx = _iota16() + i0
      plsc.store_scatter(val16_v, [ridx, jnp.zeros((L,), jnp.int32)], wn)
      plsc.store_scatter(val16_v, [ridx, jnp.full((L,), 1, jnp.int32)],
                         ewv * wn)
      plsc.store_scatter(val16_v, [ridx, jnp.full((L,), 2, jnp.int32)],
                         eav * wn)
      plsc.store_scatter(val16_v, [ridx, jnp.full((L,), 3, jnp.int32)], ones)
    descs = [
        pltpu.async_copy(val16_v.at[pl.ds(j * 128, 128)],
                         acc.at[dst2_v.at[j]], sem, add=True)
        for j in range(BS // 128)
    ]
    for d in descs:
      d.wait()
    return carry

  lax.fori_loop(0, E_PAD // 2 // NS // BS, seg_batch, 0)
  plsc.subcore_barrier()
  for k in range(3):
    r0 = s * RPTS + k * CHS
    pltpu.sync_copy(acc.at[pl.ds(r0, CHS)], st_v)
    pltpu.sync_copy(st_v, segp_hbm.at[c].at[pl.ds(r0, CHS)])


_sc_spmm = pl.kernel(
    _sc_spmm_body,
    out_type=[
        jax.ShapeDtypeStruct((NCG, NPS, CW), jnp.float32),
        jax.ShapeDtypeStruct((NC, NPS, CW), jnp.float32),
    ],
    mesh=_MESH,
    scratch_types=[
        pltpu.VMEM((BS // 128, 128), jnp.int32),
        pltpu.VMEM((BS // 128, 128), jnp.int32),
        pltpu.VMEM((BS,), jnp.float32),
        pltpu.VMEM((BS,), jnp.float32),
        pltpu.VMEM((BS,), jnp.float32),
        pltpu.VMEM((BS, CW), jnp.float32),
        pltpu.VMEM((BS, CW), jnp.float32),
        pltpu.VMEM((CHS, CW), jnp.float32),
        pltpu.VMEM((BS // 128, 128), jnp.int32),
        pltpu.VMEM((BS // 128, 128), jnp.int32),
        pltpu.VMEM((BS,), jnp.float32),
        pltpu.VMEM((BS, CW), jnp.float32),
        pltpu.VMEM_SHARED((NPS, CW), jnp.float32),
        pltpu.SemaphoreType.DMA,
        pltpu.SemaphoreType.DMA,
        pltpu.SemaphoreType.DMA,
        pltpu.SemaphoreType.DMA,
        pltpu.SemaphoreType.DMA,
        pltpu.SemaphoreType.DMA,
        pltpu.SemaphoreType.DMA,
        pltpu.SemaphoreType.DMA,
    ],
    compiler_params=_SC_PARAMS,
)


_RB = 2000
_GRID = N // _RB


def _tc0_body(x_ref, w0t_ref, b0_ref, hs_ref):
  v = jnp.dot(x_ref[...], w0t_ref[...],
              preferred_element_type=jnp.float32) + b0_ref[...]
  h = jnp.where(v >= 0.0, v, 0.01 * v)
  for k in range(NCG):
    hs_ref[k] = h[:, k * CW:(k + 1) * CW]


def _tc0(x, w0t, b0):
  return pl.pallas_call(
      _tc0_body,
      grid=(_GRID,),
      in_specs=[
          pl.BlockSpec((_RB, FIN), lambda i: (i, 0)),
          pl.BlockSpec((FIN, D), lambda i: (0, 0)),
          pl.BlockSpec((1, D), lambda i: (0, 0)),
      ],
      out_specs=pl.BlockSpec((NCG, _RB, CW), lambda i: (0, i, 0)),
      out_shape=jax.ShapeDtypeStruct((NCG, N, CW), jnp.float32),
  )(x, w0t, b0)


def _tc_dense_body(g_ref, hs_ref, seg_ref,
                   wat_ref, wbt_ref, wew_ref, wea_ref,
                   l2t_ref, b2_ref, l3t_ref, b3_ref,
                   wih_ref, bih_ref, whh_ref, bhh_ref,
                   o_ref):
  seg = seg_ref[0][:, :4] + seg_ref[1][:, :4]
  s_wn = seg[:, 0:1]
  s_ew = seg[:, 1:2]
  s_ea = seg[:, 2:3]
  cnt = seg[:, 3:4]
  g = jnp.concatenate([g_ref[k] for k in range(NCG)], axis=1)
  h = jnp.concatenate([hs_ref[k] for k in range(NCG)], axis=1)

  dot = functools.partial(jnp.dot, preferred_element_type=jnp.float32)
  sums = (dot(h * s_wn, wat_ref[...]) + dot(g, wbt_ref[...])
          + s_ew * wew_ref[...] + s_ea * wea_ref[...])
  agg = sums / jnp.maximum(cnt, 1.0)
  m = dot(agg, l2t_ref[...]) + b2_ref[...]
  m = jnp.maximum(m, 0.0) + jnp.log1p(jnp.exp(-jnp.abs(m))) - 0.6931471805599453
  m = dot(m, l3t_ref[...]) + b3_ref[...]
  gi = dot(m, wih_ref[...]) + bih_ref[...]
  gh = dot(h, whh_ref[...]) + bhh_ref[...]
  r = jax.nn.sigmoid(gi[:, :D] + gh[:, :D])
  zt = jax.nn.sigmoid(gi[:, D:2 * D] + gh[:, D:2 * D])
  ng = jnp.tanh(gi[:, 2 * D:] + r * gh[:, 2 * D:])
  hn = (1.0 - zt) * ng + zt * h
  for k in range(NCG):
    o_ref[k] = hn[:, k * CW:(k + 1) * CW]


def _tc_dense(g2, hs, segp, wat, wbt, wew, wea, l2t, b2, l3t, b3,
              wih, bih, whh, bhh):
  full = lambda shape: pl.BlockSpec(shape, lambda i, _s=shape: tuple(0 for _ in _s))
  return pl.pallas_call(
      _tc_dense_body,
      grid=(_GRID,),
      in_specs=[
          pl.BlockSpec((NCG, _RB, CW), lambda i: (0, i, 0)),
          pl.BlockSpec((NCG, _RB, CW), lambda i: (0, i, 0)),
          pl.BlockSpec((NC, _RB, CW), lambda i: (0, i, 0)),
          full((D, 2 * D)), full((D, 2 * D)), full((1, 2 * D)),
          full((1, 2 * D)),
          full((2 * D, 2 * D)), full((1, 2 * D)), full((2 * D, D)),
          full((1, D)),
          full((D, 3 * D)), full((1, 3 * D)), full((D, 3 * D)),
          full((1, 3 * D)),
      ],
      out_specs=pl.BlockSpec((NCG, _RB, CW), lambda i: (0, i, 0)),
      out_shape=jax.ShapeDtypeStruct((NCG, N, CW), jnp.float32),
  )(g2, hs, segp, wat, wbt, wew, wea, l2t, b2, l3t, b3, wih, bih, whh, bhh)


def kernel(x, edge_index, edge_weight, edge_attr, z, W0, b0, lin1_W,
           lin2_W, lin2_b, lin3_W, lin3_b, bn_gamma, bn_beta,
           gru_Wih, gru_Whh, gru_bih, gru_bhh):
  f32 = jnp.float32
  src = edge_index[0]
  dst = edge_index[1]
  npad = E_PAD - E
  srcp = jnp.concatenate([src, jnp.zeros((npad,), jnp.int32)])
  dstp = jnp.concatenate([dst, jnp.full((npad,), N, jnp.int32)])
  srcp2 = srcp.reshape(E_PAD // 128, 128)
  dstp2 = dstp.reshape(E_PAD // 128, 128)
  ewp = jnp.concatenate([edge_weight, jnp.full((npad,), 20.0, f32)])
  eap = jnp.concatenate([edge_attr[:, 0], jnp.zeros((npad,), f32)])
  zpad = jnp.concatenate([z[:, 0], jnp.zeros((NZ - N,), f32)])

  w0t = W0.T
  b0r = b0.reshape(1, D)
  wat = lin1_W[:, :D].T
  wbt = lin1_W[:, D:2 * D].T
  wew = lin1_W[:, 2 * D].reshape(1, 2 * D)
  wea = lin1_W[:, 2 * D + 1].reshape(1, 2 * D)
  l2t = lin2_W.T
  b2r = lin2_b.reshape(1, 2 * D)
  l3t = lin3_W.T
  b3r = lin3_b.reshape(1, D)
  wih = gru_Wih.T
  bih = gru_bih.reshape(1, 3 * D)
  whh = gru_Whh.T
  bhh = gru_bhh.reshape(1, 3 * D)

  zzv = jnp.zeros((BS, CW), f32)
  zz32 = jnp.zeros((CHS, CW), f32)

  part = _sc_pass1(srcp2, dstp2, ewp, zpad).reshape(NW, 128)
  s1 = jnp.sum(part[:, :L])
  s2 = jnp.sum(part[:, L:2 * L])
  mu = s1 / E
  var = s2 / E - mu * mu
  a = bn_gamma[0] / jnp.sqrt(var + 1e-5)
  b_ = bn_beta[0] - mu * a
  ab = jnp.concatenate([jnp.full((L,), a, f32), jnp.full((L,), b_, f32)])

  wn = _sc_pass2(srcp2, dstp2, ewp, zpad, ab)

  hs = _tc0(x, w0t, b0r)

  for _ in range(3):
    g2, segp = _sc_spmm(srcp2, dstp2, wn, ewp, eap, hs, zz32, zzv)
    hs = _tc_dense(g2, hs, segp, wat, wbt, wew, wea,
                   l2t, b2r, l3t, b3r, wih, bih, whh, bhh)

  return jnp.concatenate([hs[k] for k in range(NCG)], axis=1)


# speedup vs baseline: 7.8708x; 1.0295x over previous
import functools
import jax
import jax.numpy as jnp
from jax import lax
from jax.experimental import pallas as pl
from jax.experimental.pallas import tpu as pltpu
from jax.experimental.pallas import tpu_sc as plsc

N = 50000
E = 800000
FIN = 19
D = 64

NC = 2
NS = 16
NW = NC * NS
L = 16

E_PAD = 819200
NZ = N + 16
NPS = 50016

EW_W = E_PAD // NW
B12 = 5120
NB12 = EW_W // B12
NG12 = B12 // L

EW_T = E_PAD // NS
BS = 1024
NBS = EW_T // BS
NGS = BS // L

RPTS = NPS // NS
CHS = RPTS // 3

B2 = 1280
NB2 = EW_W // B2
NG2 = B2 // L

CW = 16
NCG = D // CW
NR = NCG // NC

_MESH = plsc.VectorSubcoreMesh(
    core_axis_name="c", subcore_axis_name="s", num_cores=NC, num_subcores=NS)

_SC_PARAMS = pltpu.CompilerParams(
    needs_layout_passes=False, use_tc_tiling_on_sc=False)


def _iota16():
  return lax.iota(jnp.int32, L)


def _read16(ref2d, g):
  row = jnp.full((L,), g // 8, jnp.int32)
  col = _iota16() + (g % 8) * L
  return plsc.load_gather(ref2d, [row, col])


def _sc_pass1_body(src2_hbm, dst2_hbm, ew_hbm, z_hbm, out_hbm,
                   z_v, src2_v, dst2_v, ew_v, res_v):
  c = lax.axis_index("c")
  s = lax.axis_index("s")
  wid = s * NC + c
  pltpu.sync_copy(z_hbm, z_v)
  base = wid * EW_W

  def batch_body(b, carry):
    s1, s2 = carry
    off = base + b * B12
    pltpu.sync_copy(src2_hbm.at[pl.ds(off // 128, B12 // 128)], src2_v)
    pltpu.sync_copy(dst2_hbm.at[pl.ds(off // 128, B12 // 128)], dst2_v)
    pltpu.sync_copy(ew_hbm.at[pl.ds(off, B12)], ew_v)

    def group_body(g, carry2):
      t1, t2 = carry2
      i0 = g * L
      sv = _read16(src2_v, g)
      dv = _read16(dst2_v, g)
      zj = plsc.load_gather(z_v, [sv])
      zi = plsc.load_gather(z_v, [dv])
      ewv = ew_v[pl.ds(i0, L)]
      dd = ewv - 0.5 * (zi + zj)
      ee = jnp.exp(-dd)
      wp = ee * ee - 2.0 * ee
      return (t1 + wp, t2 + wp * wp)

    return lax.fori_loop(0, NG12, group_body, (s1, s2))

  z16 = jnp.zeros((L,), jnp.float32)
  s1, s2 = lax.fori_loop(0, NB12, batch_body, (z16, z16))
  res_v[pl.ds(0, L)] = s1
  res_v[pl.ds(L, L)] = s2
  pltpu.sync_copy(res_v, out_hbm.at[pl.ds(wid * 128, 128)])


_sc_pass1 = pl.kernel(
    _sc_pass1_body,
    out_type=jax.ShapeDtypeStruct((NW * 128,), jnp.float32),
    mesh=_MESH,
    scratch_types=[
        pltpu.VMEM((NZ,), jnp.float32),
        pltpu.VMEM((B12 // 128, 128), jnp.int32),
        pltpu.VMEM((B12 // 128, 128), jnp.int32),
        pltpu.VMEM((B12,), jnp.float32),
        pltpu.VMEM((128,), jnp.float32),
    ],
    compiler_params=_SC_PARAMS,
)


def _sc_pass2_body(src2_hbm, dst2_hbm, ew_hbm, z_hbm, ab_hbm,
                   wn_hbm,
                   z_v, src2_v, dst2_v, ew_v, wn_v, ab_v):
  c = lax.axis_index("c")
  s = lax.axis_index("s")
  wid = s * NC + c
  pltpu.sync_copy(z_hbm, z_v)
  pltpu.sync_copy(ab_hbm, ab_v)

  av = ab_v[pl.ds(0, L)]
  bv = ab_v[pl.ds(L, L)]
  base = wid * EW_W

  def batch_body(b, carry):
    off = base + b * B2
    pltpu.sync_copy(src2_hbm.at[pl.ds(off // 128, B2 // 128)], src2_v)
    pltpu.sync_copy(dst2_hbm.at[pl.ds(off // 128, B2 // 128)], dst2_v)
    pltpu.sync_copy(ew_hbm.at[pl.ds(off, B2)], ew_v)

    @plsc.parallel_loop(0, NG2, unroll=2)
    def _wn_group(g):
      i0 = g * L
      sv = _read16(src2_v, g)
      dv = _read16(dst2_v, g)
      zj = plsc.load_gather(z_v, [sv])
      zi = plsc.load_gather(z_v, [dv])
      ewv = ew_v[pl.ds(i0, L)]
      dd = ewv - 0.5 * (zi + zj)
      ee = jnp.exp(-dd)
      wp = ee * ee - 2.0 * ee
      wn_v[pl.ds(i0, L)] = av * wp + bv
    pltpu.sync_copy(wn_v, wn_hbm.at[pl.ds(off, B2)])
    return carry

  lax.fori_loop(0, NB2, batch_body, 0)


_sc_pass2 = pl.kernel(
    _sc_pass2_body,
    out_type=jax.ShapeDtypeStruct((E_PAD,), jnp.float32),
    mesh=_MESH,
    scratch_types=[
        pltpu.VMEM((NZ,), jnp.float32),
        pltpu.VMEM((B2 // 128, 128), jnp.int32),
        pltpu.VMEM((B2 // 128, 128), jnp.int32),
        pltpu.VMEM((B2,), jnp.float32),
        pltpu.VMEM((B2,), jnp.float32),
        pltpu.VMEM((2 * L,), jnp.float32),
    ],
    compiler_params=_SC_PARAMS,
)


def _sc_spmm_body(src2_hbm, dst2_hbm, wn_hbm, ew_hbm, ea_hbm, tab_hbm,
                  zz32_hbm, zzv_hbm,
                  g_hbm, segp_hbm,
                  src2_v, dst2_v, wn_v, ew_v, ea_v, rows_v, val16_v, st_v,
                  src2b_v, dst2b_v, wnb_v, rowsb_v,
                  acc, sem, sem_gb, sem_sa, sem_sb,
                  sem_wa, sem_da, sem_wb, sem_db):
  c = lax.axis_index("c")
  s = lax.axis_index("s")
  base = s * EW_T
  pltpu.sync_copy(zz32_hbm, st_v)

  def _scale_rows(rows_ref, wn_ref):
    @plsc.parallel_loop(0, NGS, unroll=4)
    def _scale(g):
      i0 = g * L
      wv = wn_ref[pl.ds(i0, L)]
      ridx = _iota16() + i0
      for col in range(CW):
        cidx = jnp.full((L,), col, jnp.int32)
        v = plsc.load_gather(rows_ref, [ridx, cidx]) * wv
        plsc.store_scatter(rows_ref, [ridx, cidx], v)

  for r in range(NR):
    grp = c * NR + r
    for k in range(3):
      pltpu.sync_copy(st_v, acc.at[pl.ds(s * RPTS + k * CHS, CHS)])
    plsc.subcore_barrier()

    def pair_body(bb, carry):
      off_a = base + (2 * bb) * BS
      off_b = off_a + BS
      pltpu.sync_copy(src2_hbm.at[pl.ds(off_a // 128, BS // 128)], src2_v)
      gds_a = [
          pltpu.async_copy(tab_hbm.at[grp].at[src2_v.at[j]],
                           rows_v.at[pl.ds(j * 128, 128)], sem)
          for j in range(BS // 128)
      ]
      wd_a = pltpu.async_copy(wn_hbm.at[pl.ds(off_a, BS)], wn_v, sem_wa)
      dd_a = pltpu.async_copy(dst2_hbm.at[pl.ds(off_a // 128, BS // 128)],
                              dst2_v, sem_da)
      pltpu.sync_copy(src2_hbm.at[pl.ds(off_b // 128, BS // 128)], src2b_v)
      gds_b = [
          pltpu.async_copy(tab_hbm.at[grp].at[src2b_v.at[j]],
                           rowsb_v.at[pl.ds(j * 128, 128)], sem_gb)
          for j in range(BS // 128)
      ]
      wd_b = pltpu.async_copy(wn_hbm.at[pl.ds(off_b, BS)], wnb_v, sem_wb)
      dd_b = pltpu.async_copy(dst2_hbm.at[pl.ds(off_b // 128, BS // 128)],
                              dst2b_v, sem_db)
      for d in gds_a:
        d.wait()
      wd_a.wait()
      _scale_rows(rows_v, wn_v)
      dd_a.wait()
      sds_a = [
          pltpu.async_copy(rows_v.at[pl.ds(j * 128, 128)],
                           acc.at[dst2_v.at[j]], sem_sa, add=True)
          for j in range(BS // 128)
      ]
      for d in gds_b:
        d.wait()
      wd_b.wait()
      _scale_rows(rowsb_v, wnb_v)
      for d in sds_a:
        d.wait()
      dd_b.wait()
      sds_b = [
          pltpu.async_copy(rowsb_v.at[pl.ds(j * 128, 128)],
                           acc.at[dst2b_v.at[j]], sem_sb, add=True)
          for j in range(BS // 128)
      ]
      for d in sds_b:
        d.wait()
      return carry

    lax.fori_loop(0, NBS // 2, pair_body, 0)
    plsc.subcore_barrier()
    for k in range(3):
      r0 = s * RPTS + k * CHS
      pltpu.sync_copy(acc.at[pl.ds(r0, CHS)], st_v)
      pltpu.sync_copy(st_v, g_hbm.at[grp].at[pl.ds(r0, CHS)])
    plsc.subcore_barrier()
    pltpu.sync_copy(zz32_hbm, st_v)

  pltpu.sync_copy(zzv_hbm, val16_v)
  for k in range(3):
    pltpu.sync_copy(st_v, acc.at[pl.ds(s * RPTS + k * CHS, CHS)])
  plsc.subcore_barrier()
  ones = jnp.full((L,), 1.0, jnp.float32)
  sbase = c * (E_PAD // 2) + s * (E_PAD // 2 // NS)

  def seg_batch(b, carry):
    off = sbase + b * BS
    lds = [
        pltpu.async_copy(dst2_hbm.at[pl.ds(off // 128, BS // 128)], dst2_v,
                         sem_da),
        pltpu.async_copy(wn_hbm.at[pl.ds(off, BS)], wn_v, sem_wa),
        pltpu.async_copy(ew_hbm.at[pl.ds(off, BS)], ew_v, sem_wb),
        pltpu.async_copy(ea_hbm.at[pl.ds(off, BS)], ea_v, sem_db),
    ]
    for d in lds:
      d.wait()

    @plsc.parallel_loop(0, NGS, unroll=4)
    def _seg_group(g):
      i0 = g * L
      wn = wn_v[pl.ds(i0, L)]
      ewv = ew_v[pl.ds(i0, L)]
      eav = ea_v[pl.ds(i0, L)]
      ridx = _iota16() + i0
      plsc.store_scatter(val16_v, [ridx, jnp.zeros((L,), jnp.int32)], wn)
      plsc.store_scatter(val16_v, [ridx, jnp.full((L,), 1, jnp.int32)],
                         ewv * wn)
      plsc.store_scatter(val16_v, [ridx, jnp.full((L,), 2, jnp.int32)],
                         eav * wn)
      plsc.store_scatter(val16_v, [ridx, jnp.full((L,), 3, jnp.int32)], ones)
    descs = [
        pltpu.async_copy(val16_v.at[pl.ds(j * 128, 128)],
                         acc.at[dst2_v.at[j]], sem, add=True)
        for j in range(BS // 128)
    ]
    for d in descs:
      d.wait()
    return carry

  lax.fori_loop(0, E_PAD // 2 // NS // BS, seg_batch, 0)
  plsc.subcore_barrier()
  for k in range(3):
    r0 = s * RPTS + k * CHS
    pltpu.sync_copy(acc.at[pl.ds(r0, CHS)], st_v)
    pltpu.sync_copy(st_v, segp_hbm.at[c].at[pl.ds(r0, CHS)])


_sc_spmm = pl.kernel(
    _sc_spmm_body,
    out_type=[
        jax.ShapeDtypeStruct((NCG, NPS, CW), jnp.float32),
        jax.ShapeDtypeStruct((NC, NPS, CW), jnp.float32),
    ],
    mesh=_MESH,
    scratch_types=[
        pltpu.VMEM((BS // 128, 128), jnp.int32),
        pltpu.VMEM((BS // 128, 128), jnp.int32),
        pltpu.VMEM((BS,), jnp.float32),
        pltpu.VMEM((BS,), jnp.float32),
        pltpu.VMEM((BS,), jnp.float32),
        pltpu.VMEM((BS, CW), jnp.float32),
        pltpu.VMEM((BS, CW), jnp.float32),
        pltpu.VMEM((CHS, CW), jnp.float32),
        pltpu.VMEM((BS // 128, 128), jnp.int32),
        pltpu.VMEM((BS // 128, 128), jnp.int32),
        pltpu.VMEM((BS,), jnp.float32),
        pltpu.VMEM((BS, CW), jnp.float32),
        pltpu.VMEM_SHARED((NPS, CW), jnp.float32),
        pltpu.SemaphoreType.DMA,
        pltpu.SemaphoreType.DMA,
        pltpu.SemaphoreType.DMA,
        pltpu.SemaphoreType.DMA,
        pltpu.SemaphoreType.DMA,
        pltpu.SemaphoreType.DMA,
        pltpu.SemaphoreType.DMA,
        pltpu.SemaphoreType.DMA,
    ],
    compiler_params=_SC_PARAMS,
)


_RB = 2000
_GRID = N // _RB


def _tc0_body(x_ref, w0t_ref, b0_ref, hs_ref):
  v = jnp.dot(x_ref[...], w0t_ref[...],
              preferred_element_type=jnp.float32) + b0_ref[...]
  h = jnp.where(v >= 0.0, v, 0.01 * v)
  for k in range(NCG):
    hs_ref[k] = h[:, k * CW:(k + 1) * CW]


def _tc0(x, w0t, b0):
  return pl.pallas_call(
      _tc0_body,
      grid=(_GRID,),
      in_specs=[
          pl.BlockSpec((_RB, FIN), lambda i: (i, 0)),
          pl.BlockSpec((FIN, D), lambda i: (0, 0)),
          pl.BlockSpec((1, D), lambda i: (0, 0)),
      ],
      out_specs=pl.BlockSpec((NCG, _RB, CW), lambda i: (0, i, 0)),
      out_shape=jax.ShapeDtypeStruct((NCG, N, CW), jnp.float32),
  )(x, w0t, b0)


def _tc_dense_body(g_ref, hs_ref, seg_ref,
                   wat_ref, wbt_ref, wew_ref, wea_ref,
                   l2t_ref, b2_ref, l3t_ref, b3_ref,
                   wih_ref, bih_ref, whh_ref, bhh_ref,
                   o_ref):
  seg = seg_ref[0][:, :4] + seg_ref[1][:, :4]
  s_wn = seg[:, 0:1]
  s_ew = seg[:, 1:2]
  s_ea = seg[:, 2:3]
  cnt = seg[:, 3:4]
  g = jnp.concatenate([g_ref[k] for k in range(NCG)], axis=1)
  h = jnp.concatenate([hs_ref[k] for k in range(NCG)], axis=1)

  dot = functools.partial(jnp.dot, preferred_element_type=jnp.float32)
  sums = (dot(h * s_wn, wat_ref[...]) + dot(g, wbt_ref[...])
          + s_ew * wew_ref[...] + s_ea * wea_ref[...])
  agg = sums / jnp.maximum(cnt, 1.0)
  m = dot(agg, l2t_ref[...]) + b2_ref[...]
  m = jnp.maximum(m, 0.0) + jnp.log1p(jnp.exp(-jnp.abs(m))) - 0.6931471805599453
  m = dot(m, l3t_ref[...]) + b3_ref[...]
  gi = dot(m, wih_ref[...]) + bih_ref[...]
  gh = dot(h, whh_ref[...]) + bhh_ref[...]
  r = jax.nn.sigmoid(gi[:, :D] + gh[:, :D])
  zt = jax.nn.sigmoid(gi[:, D:2 * D] + gh[:, D:2 * D])
  ng = jnp.tanh(gi[:, 2 * D:] + r * gh[:, 2 * D:])
  hn = (1.0 - zt) * ng + zt * h
  for k in range(NCG):
    o_ref[k] = hn[:, k * CW:(k + 1) * CW]


def _tc_dense(g2, hs, segp, wat, wbt, wew, wea, l2t, b2, l3t, b3,
              wih, bih, whh, bhh):
  full = lambda shape: pl.BlockSpec(shape, lambda i, _s=shape: tuple(0 for _ in _s))
  return pl.pallas_call(
      _tc_dense_body,
      grid=(_GRID,),
      in_specs=[
          pl.BlockSpec((NCG, _RB, CW), lambda i: (0, i, 0)),
          pl.BlockSpec((NCG, _RB, CW), lambda i: (0, i, 0)),
          pl.BlockSpec((NC, _RB, CW), lambda i: (0, i, 0)),
          full((D, 2 * D)), full((D, 2 * D)), full((1, 2 * D)),
          full((1, 2 * D)),
          full((2 * D, 2 * D)), full((1, 2 * D)), full((2 * D, D)),
          full((1, D)),
          full((D, 3 * D)), full((1, 3 * D)), full((D, 3 * D)),
          full((1, 3 * D)),
      ],
      out_specs=pl.BlockSpec((NCG, _RB, CW), lambda i: (0, i, 0)),
      out_shape=jax.ShapeDtypeStruct((NCG, N, CW), jnp.float32),
  )(g2, hs, segp, wat, wbt, wew, wea, l2t, b2, l3t, b3, wih, bih, whh, bhh)


def kernel(x, edge_index, edge_weight, edge_attr, z, W0, b0, lin1_W,
           lin2_W, lin2_b, lin3_W, lin3_b, bn_gamma, bn_beta,
           gru_Wih, gru_Whh, gru_bih, gru_bhh):
  f32 = jnp.float32
  src = edge_index[0]
  dst = edge_index[1]
  npad = E_PAD - E
  srcp = jnp.concatenate([src, jnp.zeros((npad,), jnp.int32)])
  dstp = jnp.concatenate([dst, jnp.full((npad,), N, jnp.int32)])
  srcp2 = srcp.reshape(E_PAD // 128, 128)
  dstp2 = dstp.reshape(E_PAD // 128, 128)
  ewp = jnp.concatenate([edge_weight, jnp.full((npad,), 20.0, f32)])
  eap = jnp.concatenate([edge_attr[:, 0], jnp.zeros((npad,), f32)])
  zpad = jnp.concatenate([z[:, 0], jnp.zeros((NZ - N,), f32)])

  w0t = W0.T
  b0r = b0.reshape(1, D)
  wat = lin1_W[:, :D].T
  wbt = lin1_W[:, D:2 * D].T
  wew = lin1_W[:, 2 * D].reshape(1, 2 * D)
  wea = lin1_W[:, 2 * D + 1].reshape(1, 2 * D)
  l2t = lin2_W.T
  b2r = lin2_b.reshape(1, 2 * D)
  l3t = lin3_W.T
  b3r = lin3_b.reshape(1, D)
  wih = gru_Wih.T
  bih = gru_bih.reshape(1, 3 * D)
  whh = gru_Whh.T
  bhh = gru_bhh.reshape(1, 3 * D)

  zzv = jnp.zeros((BS, CW), f32)
  zz32 = jnp.zeros((CHS, CW), f32)

  part = _sc_pass1(srcp2, dstp2, ewp, zpad).reshape(NW, 128)
  s1 = jnp.sum(part[:, :L])
  s2 = jnp.sum(part[:, L:2 * L])
  mu = s1 / E
  var = s2 / E - mu * mu
  a = bn_gamma[0] / jnp.sqrt(var + 1e-5)
  b_ = bn_beta[0] - mu * a
  ab = jnp.concatenate([jnp.full((L,), a, f32), jnp.full((L,), b_, f32)])

  wn = _sc_pass2(srcp2, dstp2, ewp, zpad, ab)

  hs = _tc0(x, w0t, b0r)

  for _ in range(3):
    g2, segp = _sc_spmm(srcp2, dstp2, wn, ewp, eap, hs, zz32, zzv)
    hs = _tc_dense(g2, hs, segp, wat, wbt, wew, wea,
                   l2t, b2r, l3t, b3r, wih, bih, whh, bhh)

  return jnp.concatenate([hs[k] for k in range(NCG)], axis=1)
